# Initial kernel scaffold; baseline (speedup 1.0000x reference)
import functools

import jax
import jax.numpy as jnp
from jax import lax
from jax.experimental import pallas as pl
from jax.experimental.pallas import tpu as pltpu
from jax.experimental.pallas import tpu_sc as plsc

U_N = 25000
I_N = 25000
D = 128
E_S = 400000
E_UI = 600000

NC = 2
NS = 16
L = 16
CH = 128

ES_P = 401408
EU_P = 602112

BSZ = 12500
BACC = 12544
DSZ = 12512

_f32 = jnp.float32
_i32 = jnp.int32


def _mesh():
    return plsc.VectorSubcoreMesh(core_axis_name="c", subcore_axis_name="s")


def _spmm_body(n_out, epad, x_hbm, rows_hbm, cols_hbm, vals_hbm, out_hbm,
               gath, colv, rowv, valv, idxv, zbuf, acc, sem):
    c = lax.axis_index("c")
    s = lax.axis_index("s")
    nb = n_out // BSZ // NC
    chunks = epad // NS // CH
    zero = jnp.zeros((L,), _f32)
    for r in range(112):
        for j in range(D // L):
            zbuf[r, pl.ds(j * L, L)] = zero

    for b in range(nb):
        row_base = (c + NC * b) * BSZ
        for k in range(7):
            pltpu.sync_copy(zbuf, acc.at[pl.ds(s * 784 + k * 112, 112)])
        plsc.subcore_barrier()

        def chunk_body(i, carry):
            base = s * (chunks * CH) + i * CH
            pltpu.sync_copy(cols_hbm.at[pl.ds(base, CH)], colv)
            pltpu.sync_copy(rows_hbm.at[pl.ds(base, CH)], rowv)
            pltpu.sync_copy(vals_hbm.at[pl.ds(base, CH)], valv)
            for g in range(CH // L):
                r16 = rowv[pl.ds(g * L, L)]
                li = r16 - row_base
                owned = (li >= 0) & (li < BSZ)
                idxv[pl.ds(g * L, L)] = jnp.where(owned, li, BSZ)
            pltpu.async_copy(x_hbm.at[colv], gath, sem).wait()

            def scale_body(t, _):
                for u in range(8):
                    e = t * 8 + u
                    v = plsc.load_gather(valv, [jnp.full((L,), e, _i32)])
                    for j in range(D // L):
                        sl = pl.ds(j * L, L)
                        gath[e, sl] = gath[e, sl] * v
                return _

            lax.fori_loop(0, CH // 8, scale_body, 0)
            pltpu.sync_copy(gath, acc.at[idxv], add=True)
            return carry

        lax.fori_loop(0, chunks, chunk_body, 0)
        plsc.subcore_barrier()
        for kk in range(2):
            k = s + kk * NS

            @pl.when(k < 25)
            def _():
                pltpu.sync_copy(acc.at[pl.ds(k * 500, 500)],
                                out_hbm.at[pl.ds(row_base + k * 500, 500)])
        plsc.subcore_barrier()


@functools.lru_cache(maxsize=None)
def _make_spmm(n_x, n_out, epad):
    scratch = [
        pltpu.VMEM((CH, D), _f32),
        pltpu.VMEM((CH,), _i32),
        pltpu.VMEM((CH,), _i32),
        pltpu.VMEM((CH,), _f32),
        pltpu.VMEM((CH,), _i32),
        pltpu.VMEM((112, D), _f32),
        pltpu.VMEM_SHARED((BACC, D), _f32),
        pltpu.SemaphoreType.DMA,
    ]
    return pl.kernel(
        functools.partial(_spmm_body, n_out, epad),
        out_type=jax.ShapeDtypeStruct((n_out, D), _f32),
        mesh=_mesh(),
        scratch_types=scratch,
    )


def _sims_body(epad, an_hbm, rows_hbm, cols_hbm, sims_hbm, part_hbm,
               gA, gB, rowv, colv, simbuf, accv, sem1, sem2):
    c = lax.axis_index("c")
    s = lax.axis_index("s")
    wid = s * NC + c
    chunks = epad // (NC * NS) // CH
    iota = lax.broadcasted_iota(_i32, (L,), 0)
    accv[...] = jnp.zeros((L,), _f32)

    def chunk_body(i, carry):
        base = wid * (chunks * CH) + i * CH
        pltpu.sync_copy(rows_hbm.at[pl.ds(base, CH)], rowv)
        pltpu.sync_copy(cols_hbm.at[pl.ds(base, CH)], colv)
        d1 = pltpu.async_copy(an_hbm.at[rowv], gA, sem1)
        d2 = pltpu.async_copy(an_hbm.at[colv], gB, sem2)
        d1.wait()
        d2.wait()
        for g in range(CH // L):
            F = jnp.zeros((L,), _f32)
            for u in range(L):
                e = g * L + u
                p = gA[e, pl.ds(0, L)] * gB[e, pl.ds(0, L)]
                for j in range(1, D // L):
                    sl = pl.ds(j * L, L)
                    p = p + gA[e, sl] * gB[e, sl]
                F = jnp.where(iota == u, jnp.sum(p), F)
            sim = (F + 1.0) * 0.5
            simbuf[pl.ds(g * L, L)] = sim
            ids = base + g * L + iota
            accv[...] = accv[...] + jnp.where(ids < E_S, sim, 0.0)
        pltpu.sync_copy(simbuf, sims_hbm.at[pl.ds(base, CH)])
        return carry

    lax.fori_loop(0, chunks, chunk_body, 0)
    pltpu.sync_copy(accv, part_hbm.at[wid])


@functools.lru_cache(maxsize=None)
def _make_sims(epad):
    scratch = [
        pltpu.VMEM((CH, D), _f32),
        pltpu.VMEM((CH, D), _f32),
        pltpu.VMEM((CH,), _i32),
        pltpu.VMEM((CH,), _i32),
        pltpu.VMEM((CH,), _f32),
        pltpu.VMEM((L,), _f32),
        pltpu.SemaphoreType.DMA,
        pltpu.SemaphoreType.DMA,
    ]
    return pl.kernel(
        functools.partial(_sims_body, epad),
        out_type=(jax.ShapeDtypeStruct((epad,), _f32),
                  jax.ShapeDtypeStruct((NC * NS, L), _f32)),
        mesh=_mesh(),
        scratch_types=scratch,
    )


def _mean_threshold(partv):
    tot = jnp.zeros((L,), _f32)
    for t in range(NC * NS):
        tot = tot + partv[t, pl.ds(0, L)]
    mean = jnp.sum(tot) * (1.0 / E_S)
    return jnp.where(mean > 0.7, 0.8, 0.0)


def _diags_body(epad, sims_hbm, rows_hbm, part_hbm, diags_hbm,
                partv, bins, simv, rowv, tmp, accb, stage):
    c = lax.axis_index("c")
    s = lax.axis_index("s")
    chunks = epad // NS // CH
    iota = lax.broadcasted_iota(_i32, (L,), 0)
    pltpu.sync_copy(part_hbm, partv)
    prun = _mean_threshold(partv)
    row_base = c * BSZ

    def zero_body(i, carry):
        bins[pl.ds(i * L, L)] = jnp.zeros((L,), _f32)
        return carry

    lax.fori_loop(0, BACC // L, zero_body, 0)

    def chunk_body(i, carry):
        base = s * (chunks * CH) + i * CH
        pltpu.sync_copy(sims_hbm.at[pl.ds(base, CH)], simv)
        pltpu.sync_copy(rows_hbm.at[pl.ds(base, CH)], rowv)
        for g in range(CH // L):
            sv = simv[pl.ds(g * L, L)]
            r16 = rowv[pl.ds(g * L, L)]
            ids = base + g * L + iota
            pruned = jnp.where(sv < prun, 0.0, sv)
            li = r16 - row_base
            owned = (li >= 0) & (li < BSZ)
            idx = jnp.where(owned, li, BSZ)
            val = jnp.where(owned & (ids < E_S), pruned, 0.0)
            plsc.addupdate_scatter(bins, [idx], val)
        return carry

    lax.fori_loop(0, chunks, chunk_body, 0)
    pltpu.sync_copy(bins, stage.at[s])
    plsc.subcore_barrier()

    def zero_acc(i, carry):
        accb[pl.ds(i * L, L)] = jnp.zeros((L,), _f32)
        return carry

    lax.fori_loop(0, 784 // L, zero_acc, 0)

    def red_body(t, carry):
        pltpu.sync_copy(stage.at[t, pl.ds(s * 784, 784)], tmp)

        def add_body(r, carry2):
            sl = pl.ds(r * L, L)
            accb[sl] = accb[sl] + tmp[sl]
            return carry2

        lax.fori_loop(0, 784 // L, add_body, 0)
        return carry

    lax.fori_loop(0, NS, red_body, 0)

    def recip_body(r, carry):
        sl = pl.ds(r * L, L)
        accb[sl] = 1.0 / (accb[sl] + 1e-7)
        return carry

    lax.fori_loop(0, 784 // L, recip_body, 0)

    @pl.when(s < 15)
    def _():
        pltpu.sync_copy(accb, diags_hbm.at[c, pl.ds(s * 784, 784)])

    @pl.when(s == 15)
    def _():
        pltpu.sync_copy(accb.at[pl.ds(0, 740)],
                        diags_hbm.at[c, pl.ds(15 * 784, 740)])


@functools.lru_cache(maxsize=None)
def _make_diags(epad):
    scratch = [
        pltpu.VMEM((NC * NS, L), _f32),
        pltpu.VMEM((BACC,), _f32),
        pltpu.VMEM((CH,), _f32),
        pltpu.VMEM((CH,), _i32),
        pltpu.VMEM((784,), _f32),
        pltpu.VMEM((784,), _f32),
        pltpu.VMEM_SHARED((NS, BACC), _f32),
    ]
    return pl.kernel(
        functools.partial(_diags_body, epad),
        out_type=jax.ShapeDtypeStruct((NC, DSZ), _f32),
        mesh=_mesh(),
        scratch_types=scratch,
    )


def _nvals_body(epad, sims_hbm, rows_hbm, diags_hbm, part_hbm, out_hbm,
                dbuf, partv, simv, rowv, outv):
    c = lax.axis_index("c")
    s = lax.axis_index("s")
    wid = s * NC + c
    chunks = epad // (NC * NS) // CH
    iota = lax.broadcasted_iota(_i32, (L,), 0)
    pltpu.sync_copy(part_hbm, partv)
    prun = _mean_threshold(partv)
    pltpu.sync_copy(diags_hbm, dbuf)

    def chunk_body(i, carry):
        base = wid * (chunks * CH) + i * CH
        pltpu.sync_copy(sims_hbm.at[pl.ds(base, CH)], simv)
        pltpu.sync_copy(rows_hbm.at[pl.ds(base, CH)], rowv)
        for g in range(CH // L):
            sv = simv[pl.ds(g * L, L)]
            r16 = rowv[pl.ds(g * L, L)]
            ids = base + g * L + iota
            pruned = jnp.where(sv < prun, 0.0, sv)
            pruned = jnp.where(ids < E_S, pruned, 0.0)
            bkt = jnp.where(r16 >= BSZ, 1, 0)
            li = r16 - bkt * BSZ
            dg = plsc.load_gather(dbuf, [bkt, li])
            outv[pl.ds(g * L, L)] = pruned * dg
        pltpu.sync_copy(outv, out_hbm.at[pl.ds(base, CH)])
        return carry

    lax.fori_loop(0, chunks, chunk_body, 0)


@functools.lru_cache(maxsize=None)
def _make_nvals(epad):
    scratch = [
        pltpu.VMEM((NC, DSZ), _f32),
        pltpu.VMEM((NC * NS, L), _f32),
        pltpu.VMEM((CH,), _f32),
        pltpu.VMEM((CH,), _i32),
        pltpu.VMEM((CH,), _f32),
    ]
    return pl.kernel(
        functools.partial(_nvals_body, epad),
        out_type=jax.ShapeDtypeStruct((epad,), _f32),
        mesh=_mesh(),
        scratch_types=scratch,
    )


_RB = 1000


def _norm_body(x_ref, o_ref):
    x = x_ref[...]
    n = jnp.sqrt(jnp.sum(x * x, axis=1, keepdims=True))
    o_ref[...] = x / jnp.maximum(n, 1e-8)


def _normalize(x):
    return pl.pallas_call(
        _norm_body,
        grid=(U_N // _RB,),
        in_specs=[pl.BlockSpec((_RB, D), lambda i: (i, 0))],
        out_specs=pl.BlockSpec((_RB, D), lambda i: (i, 0)),
        out_shape=jax.ShapeDtypeStruct((U_N, D), _f32),
    )(x)


def _ego_body(u_ref, l1_ref, l2_ref, l3_ref, it_ref, ego_ref, sv_ref):
    i = pl.program_id(0)
    sv = (l1_ref[...] + l2_ref[...] + l3_ref[...]) * (1.0 / 3.0)
    sv_ref[...] = sv

    @pl.when(i < U_N // _RB)
    def _():
        ego_ref[...] = u_ref[...] + sv

    @pl.when(i >= U_N // _RB)
    def _():
        ego_ref[...] = it_ref[...]


def _ego(user_emb, l1, l2, l3, item_emb):
    nu = U_N // _RB
    user_map = lambda i: (jnp.minimum(i, nu - 1), 0)
    item_map = lambda i: (jnp.maximum(i - nu, 0), 0)
    return pl.pallas_call(
        _ego_body,
        grid=((U_N + I_N) // _RB,),
        in_specs=[
            pl.BlockSpec((_RB, D), user_map),
            pl.BlockSpec((_RB, D), user_map),
            pl.BlockSpec((_RB, D), user_map),
            pl.BlockSpec((_RB, D), user_map),
            pl.BlockSpec((_RB, D), item_map),
        ],
        out_specs=(
            pl.BlockSpec((_RB, D), lambda i: (i, 0)),
            pl.BlockSpec((_RB, D), user_map),
        ),
        out_shape=(jax.ShapeDtypeStruct((U_N + I_N, D), _f32),
                   jax.ShapeDtypeStruct((U_N, D), _f32)),
    )(user_emb, l1, l2, l3, item_emb)


def _final_body(e0u_ref, e1u_ref, e2u_ref, e0i_ref, e1i_ref, e2i_ref,
                sv_ref, g1_ref, g2_ref, user_ref, item_ref):
    uv1 = (e0u_ref[...] + e1u_ref[...] + e2u_ref[...]) * (1.0 / 3.0)
    iv1 = (e0i_ref[...] + e1i_ref[...] + e2i_ref[...]) * (1.0 / 3.0)
    item_ref[...] = iv1
    sv = sv_ref[...]
    dn = (((1,), (1,)), ((), ()))
    z = (lax.dot_general(uv1, g1_ref[...], dn, preferred_element_type=_f32)
         + lax.dot_general(sv, g2_ref[...], dn, preferred_element_type=_f32))
    g = jax.nn.sigmoid(z)
    user_ref[...] = g * sv + (1.0 - g) * uv1


def _final(ego, e1, e2, sview, g1, g2):
    nu = U_N // _RB
    umap = lambda i: (i, 0)
    imap = lambda i: (i + nu, 0)
    wmap = lambda i: (0, 0)
    return pl.pallas_call(
        _final_body,
        grid=(nu,),
        in_specs=[
            pl.BlockSpec((_RB, D), umap),
            pl.BlockSpec((_RB, D), umap),
            pl.BlockSpec((_RB, D), umap),
            pl.BlockSpec((_RB, D), imap),
            pl.BlockSpec((_RB, D), imap),
            pl.BlockSpec((_RB, D), imap),
            pl.BlockSpec((_RB, D), umap),
            pl.BlockSpec((D, D), wmap),
            pl.BlockSpec((D, D), wmap),
        ],
        out_specs=(pl.BlockSpec((_RB, D), umap),
                   pl.BlockSpec((_RB, D), umap)),
        out_shape=(jax.ShapeDtypeStruct((U_N, D), _f32),
                   jax.ShapeDtypeStruct((U_N, D), _f32)),
    )(ego, e1, e2, ego, e1, e2, sview, g1, g2)


def kernel(user_emb, item_emb, gate1_W, gate2_W, social_index, social_values,
           norm_index, norm_values):
    s_rows = jnp.pad(social_index[0], (0, ES_P - E_S))
    s_cols = jnp.pad(social_index[1], (0, ES_P - E_S))
    s_vals = jnp.pad(social_values, (0, ES_P - E_S))
    u_rows = jnp.pad(norm_index[0], (0, EU_P - E_UI))
    u_cols = jnp.pad(norm_index[1], (0, EU_P - E_UI))
    u_vals = jnp.pad(norm_values, (0, EU_P - E_UI))

    spmm_s = _make_spmm(U_N, U_N, ES_P)
    spmm_ui = _make_spmm(U_N + I_N, U_N + I_N, EU_P)

    u_agg = spmm_s(user_emb, s_rows, s_cols, s_vals)
    a_norm = _normalize(u_agg)
    sims, parts = _make_sims(ES_P)(a_norm, s_rows, s_cols)
    diags = _make_diags(ES_P)(sims, s_rows, parts)
    nvals = _make_nvals(ES_P)(sims, s_rows, diags, parts)
    l1 = spmm_s(user_emb, s_rows, s_cols, nvals)
    l2 = spmm_s(l1, s_rows, s_cols, nvals)
    l3 = spmm_s(l2, s_rows, s_cols, nvals)
    ego, sview = _ego(user_emb, l1, l2, l3, item_emb)
    e1 = spmm_ui(ego, u_rows, u_cols, u_vals)
    e2 = spmm_ui(e1, u_rows, u_cols, u_vals)
    return _final(ego, e1, e2, sview, gate1_W, gate2_W)


# speedup vs baseline: 1.8379x; 1.8379x over previous
import functools

import jax
import jax.numpy as jnp
from jax import lax
from jax.experimental import pallas as pl
from jax.experimental.pallas import tpu as pltpu
from jax.experimental.pallas import tpu_sc as plsc

U_N = 25000
I_N = 25000
D = 128
E_S = 400000
E_UI = 600000

NC = 2
NS = 16
L = 16
CH = 128

ES_P = 401408
EU_P = 602112

BSTR = 12544
BACC = 12800
DUMMY = 12600
FCH = 784
RSL = 800

_f32 = jnp.float32
_i32 = jnp.int32


def _mesh():
    return plsc.VectorSubcoreMesh(core_axis_name="c", subcore_axis_name="s")


def _spmm_body(n_out, epad, x_hbm, rows_hbm, cols_hbm, vals_hbm, out_hbm,
               gath, colv, rowv, valv, idxv, acc, sem):
    c = lax.axis_index("c")
    s = lax.axis_index("s")
    nbk = -(-n_out // BSTR)
    nb = nbk // NC
    chunks = epad // NS // CH
    zero = jnp.zeros((L,), _f32)

    for b in range(nb):
        bucket = b * NC
        row_base = (c + bucket) * BSTR

        def zb_body(i, carry):
            for j in range(D // L):
                gath[i, pl.ds(j * L, L)] = zero
            return carry

        lax.fori_loop(0, 128, zb_body, 0)
        for k in range(6):
            pltpu.sync_copy(gath, acc.at[pl.ds(s * RSL + k * 128, 128)])
        pltpu.sync_copy(gath.at[pl.ds(0, RSL - 6 * 128)],
                        acc.at[pl.ds(s * RSL + 6 * 128, RSL - 6 * 128)])
        plsc.subcore_barrier()

        def chunk_body(i, carry):
            base = s * (chunks * CH) + i * CH
            pltpu.sync_copy(cols_hbm.at[pl.ds(base, CH)], colv)
            pltpu.sync_copy(rows_hbm.at[pl.ds(base, CH)], rowv)
            pltpu.sync_copy(vals_hbm.at[pl.ds(base, CH)], valv)
            for g in range(CH // L):
                r16 = rowv[pl.ds(g * L, L)]
                li = r16 - row_base
                owned = (li >= 0) & (li < BSTR)
                idxv[pl.ds(g * L, L)] = jnp.where(owned, li, DUMMY)
            pltpu.async_copy(x_hbm.at[colv], gath, sem).wait()

            def scale_body(t, _):
                for u in range(8):
                    e = t * 8 + u
                    v = plsc.load_gather(valv, [jnp.full((L,), e, _i32)])
                    for j in range(D // L):
                        sl = pl.ds(j * L, L)
                        gath[e, sl] = gath[e, sl] * v
                return _

            lax.fori_loop(0, CH // 8, scale_body, 0)
            pltpu.sync_copy(gath, acc.at[idxv], add=True)
            return carry

        lax.fori_loop(0, chunks, chunk_body, 0)
        plsc.subcore_barrier()
        short_c = (nbk - 1) % NC
        is_short_b = (b == (nbk - 1) // NC)
        tail = n_out - (nbk - 1) * BSTR - 15 * FCH

        def _full_flush():
            pltpu.sync_copy(acc.at[pl.ds(s * FCH, FCH)],
                            out_hbm.at[pl.ds(row_base + s * FCH, FCH)])

        if is_short_b:
            @pl.when((c != short_c) | (s < 15))
            def _():
                _full_flush()

            @pl.when((c == short_c) & (s == 15))
            def _():
                pltpu.sync_copy(acc.at[pl.ds(15 * FCH, tail)],
                                out_hbm.at[pl.ds(row_base + 15 * FCH, tail)])
        else:
            _full_flush()
        plsc.subcore_barrier()


@functools.lru_cache(maxsize=None)
def _make_spmm(n_x, n_out, epad):
    scratch = [
        pltpu.VMEM((CH, D), _f32),
        pltpu.VMEM((CH,), _i32),
        pltpu.VMEM((CH,), _i32),
        pltpu.VMEM((CH,), _f32),
        pltpu.VMEM((CH,), _i32),
        pltpu.VMEM_SHARED((BACC, D), _f32),
        pltpu.SemaphoreType.DMA,
    ]
    return pl.kernel(
        functools.partial(_spmm_body, n_out, epad),
        out_type=jax.ShapeDtypeStruct((n_out, D), _f32),
        mesh=_mesh(),
        scratch_types=scratch,
        compiler_params=pltpu.CompilerParams(needs_layout_passes=False, use_tc_tiling_on_sc=False),
    )


def _sims_body(epad, an_hbm, rows_hbm, cols_hbm, sims_hbm, part_hbm,
               gA, gB, rowv, colv, simbuf, accv, sem1, sem2):
    c = lax.axis_index("c")
    s = lax.axis_index("s")
    wid = s * NC + c
    chunks = epad // (NC * NS) // CH
    iota = lax.broadcasted_iota(_i32, (L,), 0)
    accv[...] = jnp.zeros((L,), _f32)

    def chunk_body(i, carry):
        base = wid * (chunks * CH) + i * CH
        pltpu.sync_copy(rows_hbm.at[pl.ds(base, CH)], rowv)
        pltpu.sync_copy(cols_hbm.at[pl.ds(base, CH)], colv)
        d1 = pltpu.async_copy(an_hbm.at[rowv], gA, sem1)
        d2 = pltpu.async_copy(an_hbm.at[colv], gB, sem2)
        d1.wait()
        d2.wait()
        for g in range(CH // L):
            F = jnp.zeros((L,), _f32)
            for u in range(L):
                e = g * L + u
                p = gA[e, pl.ds(0, L)] * gB[e, pl.ds(0, L)]
                for j in range(1, D // L):
                    sl = pl.ds(j * L, L)
                    p = p + gA[e, sl] * gB[e, sl]
                F = jnp.where(iota == u, jnp.sum(p), F)
            sim = (F + 1.0) * 0.5
            simbuf[pl.ds(g * L, L)] = sim
            ids = base + g * L + iota
            accv[...] = accv[...] + jnp.where(ids < E_S, sim, 0.0)
        pltpu.sync_copy(simbuf, sims_hbm.at[pl.ds(base, CH)])
        return carry

    lax.fori_loop(0, chunks, chunk_body, 0)
    pltpu.sync_copy(accv, part_hbm.at[wid])


@functools.lru_cache(maxsize=None)
def _make_sims(epad):
    scratch = [
        pltpu.VMEM((CH, D), _f32),
        pltpu.VMEM((CH, D), _f32),
        pltpu.VMEM((CH,), _i32),
        pltpu.VMEM((CH,), _i32),
        pltpu.VMEM((CH,), _f32),
        pltpu.VMEM((L,), _f32),
        pltpu.SemaphoreType.DMA,
        pltpu.SemaphoreType.DMA,
    ]
    return pl.kernel(
        functools.partial(_sims_body, epad),
        out_type=(jax.ShapeDtypeStruct((epad,), _f32),
                  jax.ShapeDtypeStruct((NC * NS, L), _f32)),
        mesh=_mesh(),
        scratch_types=scratch,
        compiler_params=pltpu.CompilerParams(needs_layout_passes=False, use_tc_tiling_on_sc=False),
    )


def _mean_threshold(partv):
    tot = jnp.zeros((L,), _f32)
    for t in range(NC * NS):
        tot = tot + partv[t, pl.ds(0, L)]
    mean = jnp.sum(tot) * (1.0 / E_S)
    return jnp.where(mean > 0.7, 0.8, 0.0)


def _diags_body(epad, sims_hbm, rows_hbm, part_hbm, diags_hbm,
                partv, bins, simv, rowv, tmp, accb, stage):
    c = lax.axis_index("c")
    s = lax.axis_index("s")
    chunks = epad // NS // CH
    iota = lax.broadcasted_iota(_i32, (L,), 0)
    pltpu.sync_copy(part_hbm, partv)
    prun = _mean_threshold(partv)
    row_base = c * BSTR

    def zero_body(i, carry):
        bins[pl.ds(i * L, L)] = jnp.zeros((L,), _f32)
        return carry

    lax.fori_loop(0, BACC // L, zero_body, 0)

    def chunk_body(i, carry):
        base = s * (chunks * CH) + i * CH
        pltpu.sync_copy(sims_hbm.at[pl.ds(base, CH)], simv)
        pltpu.sync_copy(rows_hbm.at[pl.ds(base, CH)], rowv)
        for g in range(CH // L):
            sv = simv[pl.ds(g * L, L)]
            r16 = rowv[pl.ds(g * L, L)]
            ids = base + g * L + iota
            pruned = jnp.where(sv < prun, 0.0, sv)
            li = r16 - row_base
            owned = (li >= 0) & (li < BSTR)
            idx = jnp.where(owned, li, DUMMY)
            val = jnp.where(owned & (ids < E_S), pruned, 0.0)
            plsc.addupdate_scatter(bins, [idx], val)
        return carry

    lax.fori_loop(0, chunks, chunk_body, 0)
    pltpu.sync_copy(bins, stage.at[s])
    plsc.subcore_barrier()

    def zero_acc(i, carry):
        accb[pl.ds(i * L, L)] = jnp.zeros((L,), _f32)
        return carry

    lax.fori_loop(0, RSL // L, zero_acc, 0)

    def red_body(t, carry):
        pltpu.sync_copy(stage.at[t, pl.ds(s * RSL, RSL)], tmp)

        def add_body(r, carry2):
            sl = pl.ds(r * L, L)
            accb[sl] = accb[sl] + tmp[sl]
            return carry2

        lax.fori_loop(0, RSL // L, add_body, 0)
        return carry

    lax.fori_loop(0, NS, red_body, 0)

    def recip_body(r, carry):
        sl = pl.ds(r * L, L)
        accb[sl] = 1.0 / (accb[sl] + 1e-7)
        return carry

    lax.fori_loop(0, RSL // L, recip_body, 0)

    @pl.when(s < 15)
    def _():
        pltpu.sync_copy(accb, diags_hbm.at[c, pl.ds(s * RSL, RSL)])

    @pl.when((s == 15) & (c == 0))
    def _():
        pltpu.sync_copy(accb.at[pl.ds(0, BSTR - 15 * RSL)],
                        diags_hbm.at[c, pl.ds(15 * RSL, BSTR - 15 * RSL)])

    @pl.when((s == 15) & (c == 1))
    def _():
        pltpu.sync_copy(accb.at[pl.ds(0, U_N - BSTR - 15 * RSL)],
                        diags_hbm.at[c, pl.ds(15 * RSL, U_N - BSTR - 15 * RSL)])


@functools.lru_cache(maxsize=None)
def _make_diags(epad):
    scratch = [
        pltpu.VMEM((NC * NS, L), _f32),
        pltpu.VMEM((BACC,), _f32),
        pltpu.VMEM((CH,), _f32),
        pltpu.VMEM((CH,), _i32),
        pltpu.VMEM((RSL,), _f32),
        pltpu.VMEM((RSL,), _f32),
        pltpu.VMEM_SHARED((NS, BACC), _f32),
    ]
    return pl.kernel(
        functools.partial(_diags_body, epad),
        out_type=jax.ShapeDtypeStruct((NC, BSTR), _f32),
        mesh=_mesh(),
        scratch_types=scratch,
        compiler_params=pltpu.CompilerParams(needs_layout_passes=False, use_tc_tiling_on_sc=False),
    )


def _nvals_body(epad, sims_hbm, rows_hbm, diags_hbm, part_hbm, out_hbm,
                dbuf, partv, simv, rowv, outv):
    c = lax.axis_index("c")
    s = lax.axis_index("s")
    wid = s * NC + c
    chunks = epad // (NC * NS) // CH
    iota = lax.broadcasted_iota(_i32, (L,), 0)
    pltpu.sync_copy(part_hbm, partv)
    prun = _mean_threshold(partv)
    pltpu.sync_copy(diags_hbm, dbuf)

    def chunk_body(i, carry):
        base = wid * (chunks * CH) + i * CH
        pltpu.sync_copy(sims_hbm.at[pl.ds(base, CH)], simv)
        pltpu.sync_copy(rows_hbm.at[pl.ds(base, CH)], rowv)
        for g in range(CH // L):
            sv = simv[pl.ds(g * L, L)]
            r16 = rowv[pl.ds(g * L, L)]
            ids = base + g * L + iota
            pruned = jnp.where(sv < prun, 0.0, sv)
            pruned = jnp.where(ids < E_S, pruned, 0.0)
            bkt = jnp.where(r16 >= BSTR, 1, 0)
            li = r16 - bkt * BSTR
            dg = plsc.load_gather(dbuf, [bkt, li])
            outv[pl.ds(g * L, L)] = pruned * dg
        pltpu.sync_copy(outv, out_hbm.at[pl.ds(base, CH)])
        return carry

    lax.fori_loop(0, chunks, chunk_body, 0)


@functools.lru_cache(maxsize=None)
def _make_nvals(epad):
    scratch = [
        pltpu.VMEM((NC, BSTR), _f32),
        pltpu.VMEM((NC * NS, L), _f32),
        pltpu.VMEM((CH,), _f32),
        pltpu.VMEM((CH,), _i32),
        pltpu.VMEM((CH,), _f32),
    ]
    return pl.kernel(
        functools.partial(_nvals_body, epad),
        out_type=jax.ShapeDtypeStruct((epad,), _f32),
        mesh=_mesh(),
        scratch_types=scratch,
        compiler_params=pltpu.CompilerParams(needs_layout_passes=False, use_tc_tiling_on_sc=False),
    )


_RB = 1000


def _norm_body(x_ref, o_ref):
    x = x_ref[...]
    n = jnp.sqrt(jnp.sum(x * x, axis=1, keepdims=True))
    o_ref[...] = x / jnp.maximum(n, 1e-8)


def _normalize(x):
    return pl.pallas_call(
        _norm_body,
        grid=(U_N // _RB,),
        in_specs=[pl.BlockSpec((_RB, D), lambda i: (i, 0))],
        out_specs=pl.BlockSpec((_RB, D), lambda i: (i, 0)),
        out_shape=jax.ShapeDtypeStruct((U_N, D), _f32),
    )(x)


def _ego_body(u_ref, l1_ref, l2_ref, l3_ref, it_ref, ego_ref, sv_ref):
    i = pl.program_id(0)
    sv = (l1_ref[...] + l2_ref[...] + l3_ref[...]) * (1.0 / 3.0)
    sv_ref[...] = sv

    @pl.when(i < U_N // _RB)
    def _():
        ego_ref[...] = u_ref[...] + sv

    @pl.when(i >= U_N // _RB)
    def _():
        ego_ref[...] = it_ref[...]


def _ego(user_emb, l1, l2, l3, item_emb):
    nu = U_N // _RB
    user_map = lambda i: (jnp.minimum(i, nu - 1), 0)
    item_map = lambda i: (jnp.maximum(i - nu, 0), 0)
    return pl.pallas_call(
        _ego_body,
        grid=((U_N + I_N) // _RB,),
        in_specs=[
            pl.BlockSpec((_RB, D), user_map),
            pl.BlockSpec((_RB, D), user_map),
            pl.BlockSpec((_RB, D), user_map),
            pl.BlockSpec((_RB, D), user_map),
            pl.BlockSpec((_RB, D), item_map),
        ],
        out_specs=(
            pl.BlockSpec((_RB, D), lambda i: (i, 0)),
            pl.BlockSpec((_RB, D), user_map),
        ),
        out_shape=(jax.ShapeDtypeStruct((U_N + I_N, D), _f32),
                   jax.ShapeDtypeStruct((U_N, D), _f32)),
    )(user_emb, l1, l2, l3, item_emb)


def _final_body(e0u_ref, e1u_ref, e2u_ref, e0i_ref, e1i_ref, e2i_ref,
                sv_ref, g1_ref, g2_ref, user_ref, item_ref):
    uv1 = (e0u_ref[...] + e1u_ref[...] + e2u_ref[...]) * (1.0 / 3.0)
    iv1 = (e0i_ref[...] + e1i_ref[...] + e2i_ref[...]) * (1.0 / 3.0)
    item_ref[...] = iv1
    sv = sv_ref[...]
    dn = (((1,), (1,)), ((), ()))
    z = (lax.dot_general(uv1, g1_ref[...], dn, preferred_element_type=_f32)
         + lax.dot_general(sv, g2_ref[...], dn, preferred_element_type=_f32))
    g = jax.nn.sigmoid(z)
    user_ref[...] = g * sv + (1.0 - g) * uv1


def _final(ego, e1, e2, sview, g1, g2):
    nu = U_N // _RB
    umap = lambda i: (i, 0)
    imap = lambda i: (i + nu, 0)
    wmap = lambda i: (0, 0)
    return pl.pallas_call(
        _final_body,
        grid=(nu,),
        in_specs=[
            pl.BlockSpec((_RB, D), umap),
            pl.BlockSpec((_RB, D), umap),
            pl.BlockSpec((_RB, D), umap),
            pl.BlockSpec((_RB, D), imap),
            pl.BlockSpec((_RB, D), imap),
            pl.BlockSpec((_RB, D), imap),
            pl.BlockSpec((_RB, D), umap),
            pl.BlockSpec((D, D), wmap),
            pl.BlockSpec((D, D), wmap),
        ],
        out_specs=(pl.BlockSpec((_RB, D), umap),
                   pl.BlockSpec((_RB, D), umap)),
        out_shape=(jax.ShapeDtypeStruct((U_N, D), _f32),
                   jax.ShapeDtypeStruct((U_N, D), _f32)),
    )(ego, e1, e2, ego, e1, e2, sview, g1, g2)


def kernel(user_emb, item_emb, gate1_W, gate2_W, social_index, social_values,
           norm_index, norm_values):
    s_rows = jnp.pad(social_index[0], (0, ES_P - E_S))
    s_cols = jnp.pad(social_index[1], (0, ES_P - E_S))
    s_vals = jnp.pad(social_values, (0, ES_P - E_S))
    u_rows = jnp.pad(norm_index[0], (0, EU_P - E_UI))
    u_cols = jnp.pad(norm_index[1], (0, EU_P - E_UI))
    u_vals = jnp.pad(norm_values, (0, EU_P - E_UI))

    spmm_s = _make_spmm(U_N, U_N, ES_P)
    spmm_ui = _make_spmm(U_N + I_N, U_N + I_N, EU_P)

    u_agg = spmm_s(user_emb, s_rows, s_cols, s_vals)
    a_norm = _normalize(u_agg)
    sims, parts = _make_sims(ES_P)(a_norm, s_rows, s_cols)
    diags = _make_diags(ES_P)(sims, s_rows, parts)
    nvals = _make_nvals(ES_P)(sims, s_rows, diags, parts)
    l1 = spmm_s(user_emb, s_rows, s_cols, nvals)
    l2 = spmm_s(l1, s_rows, s_cols, nvals)
    l3 = spmm_s(l2, s_rows, s_cols, nvals)
    ego, sview = _ego(user_emb, l1, l2, l3, item_emb)
    e1 = spmm_ui(ego, u_rows, u_cols, u_vals)
    e2 = spmm_ui(e1, u_rows, u_cols, u_vals)
    return _final(ego, e1, e2, sview, gate1_W, gate2_W)


# speedup vs baseline: 3.5340x; 1.9228x over previous
import functools

import jax
import jax.numpy as jnp
from jax import lax
from jax.experimental import pallas as pl
from jax.experimental.pallas import tpu as pltpu
from jax.experimental.pallas import tpu_sc as plsc

U_N = 25000
I_N = 25000
D = 128
E_S = 400000
E_UI = 600000

NC = 2
NS = 16
L = 16
CH = 128

ES_P = 401408
EU_P = 602112

BSTR = 12544
BACC = 12800
DUMMY = 12600
FCH = 784
RSL = 800

NBK_S = 2
NBK_UI = 4
NW = NC * NS
CAP_S = ES_P // NW + CH
CAP_UI = EU_P // NW + CH

_f32 = jnp.float32
_i32 = jnp.int32


def _mesh():
    return plsc.VectorSubcoreMesh(core_axis_name="c", subcore_axis_name="s")


def _sc_params():
    return pltpu.CompilerParams(needs_layout_passes=False,
                                use_tc_tiling_on_sc=False)


def _iota():
    return lax.broadcasted_iota(_i32, (L,), 0)


def _part_body(epad, ereal, nbk, cap, rows_hbm, cols_hbm, vals_hbm,
               rec_hbm, vslot_hbm, counts_hbm, *scr):
    c = lax.axis_index("c")
    s = lax.axis_index("s")
    wid = s * NC + c
    et = epad // NW
    chunks = et // CH
    iota = _iota()
    rowv, colv, valv, countv = scr[0], scr[1], scr[2], scr[3]
    st_r = scr[4:4 + nbk]
    st_c = scr[4 + nbk:4 + 2 * nbk]
    st_v = scr[4 + 2 * nbk:4 + 3 * nbk]
    ebase = wid * et

    def chunk_body(i, carry):
        fills = list(carry[0:nbk])
        curs = list(carry[nbk:2 * nbk])
        ecnt = list(carry[2 * nbk:3 * nbk])
        base = ebase + i * CH
        pltpu.sync_copy(rows_hbm.at[pl.ds(base, CH)], rowv)
        pltpu.sync_copy(cols_hbm.at[pl.ds(base, CH)], colv)
        pltpu.sync_copy(vals_hbm.at[pl.ds(base, CH)], valv)
        for g in range(CH // L):
            sl = pl.ds(g * L, L)
            r16 = rowv[sl]
            c16 = colv[sl]
            v16 = valv[sl]
            real = (base + g * L + iota) < ereal
            for b in range(nbk):
                m = real & (r16 >= b * BSTR) & (r16 < (b + 1) * BSTR)
                plsc.store_compressed(st_r[b].at[pl.ds(fills[b], L)], r16,
                                      mask=m)
                plsc.store_compressed(st_c[b].at[pl.ds(fills[b], L)], c16,
                                      mask=m)
                plsc.store_compressed(st_v[b].at[pl.ds(fills[b], L)], v16,
                                      mask=m)
                pc = jnp.max(plsc.all_reduce_population_count(m))
                fills[b] = fills[b] + pc
                ecnt[b] = ecnt[b] + pc
        for b in range(nbk):
            do = fills[b] >= CH
            rb = (wid * nbk + b) * (cap * 2)
            vb = (wid * nbk + b) * cap

            @pl.when(do)
            def _(b=b, rb=rb, vb=vb, cur=curs[b]):
                pltpu.sync_copy(st_r[b].at[pl.ds(0, CH)],
                                rec_hbm.at[pl.ds(rb + cur * (2 * CH), CH)])
                pltpu.sync_copy(st_c[b].at[pl.ds(0, CH)],
                                rec_hbm.at[pl.ds(rb + cur * (2 * CH) + CH,
                                                 CH)])
                pltpu.sync_copy(st_v[b].at[pl.ds(0, CH)],
                                vslot_hbm.at[pl.ds(vb + cur * CH, CH)])
                for j in range(CH // L):
                    lo = pl.ds(j * L, L)
                    hi = pl.ds(CH + j * L, L)
                    st_r[b][lo] = st_r[b][hi]
                    st_c[b][lo] = st_c[b][hi]
                    st_v[b][lo] = st_v[b][hi]

            fills[b] = jnp.where(do, fills[b] - CH, fills[b])
            curs[b] = jnp.where(do, curs[b] + 1, curs[b])
        return tuple(fills) + tuple(curs) + tuple(ecnt)

    z = jnp.int32(0)
    res = lax.fori_loop(0, chunks, chunk_body, (z,) * (3 * nbk))
    fills = res[0:nbk]
    curs = res[nbk:2 * nbk]
    ecnt = res[2 * nbk:3 * nbk]
    cv = jnp.zeros((L,), _i32)
    for b in range(nbk):
        for j in range(CH // L):
            sl = pl.ds(j * L, L)
            m = (j * L + iota) >= fills[b]
            st_r[b][sl] = jnp.where(m, b * BSTR, st_r[b][sl])
            st_c[b][sl] = jnp.where(m, 0, st_c[b][sl])
            st_v[b][sl] = jnp.where(m, 0.0, st_v[b][sl])
        rb = (wid * nbk + b) * (cap * 2)
        vb = (wid * nbk + b) * cap
        pltpu.sync_copy(st_r[b].at[pl.ds(0, CH)],
                        rec_hbm.at[pl.ds(rb + curs[b] * (2 * CH), CH)])
        pltpu.sync_copy(st_c[b].at[pl.ds(0, CH)],
                        rec_hbm.at[pl.ds(rb + curs[b] * (2 * CH) + CH, CH)])
        pltpu.sync_copy(st_v[b].at[pl.ds(0, CH)],
                        vslot_hbm.at[pl.ds(vb + curs[b] * CH, CH)])
        cv = jnp.where(iota == b, ecnt[b], cv)
    countv[...] = cv
    pltpu.sync_copy(countv, counts_hbm.at[wid])


@functools.lru_cache(maxsize=None)
def _make_part(epad, ereal, nbk, cap):
    scratch = ([
        pltpu.VMEM((CH,), _i32),
        pltpu.VMEM((CH,), _i32),
        pltpu.VMEM((CH,), _f32),
        pltpu.VMEM((L,), _i32),
    ] + [pltpu.VMEM((2 * CH,), _i32) for _ in range(nbk)]
      + [pltpu.VMEM((2 * CH,), _i32) for _ in range(nbk)]
      + [pltpu.VMEM((2 * CH,), _f32) for _ in range(nbk)])
    nslot = NW * nbk
    return pl.kernel(
        functools.partial(_part_body, epad, ereal, nbk, cap),
        out_type=(jax.ShapeDtypeStruct((nslot * cap * 2,), _i32),
                  jax.ShapeDtypeStruct((nslot * cap,), _f32),
                  jax.ShapeDtypeStruct((NW, L), _i32)),
        mesh=_mesh(),
        scratch_types=scratch,
        compiler_params=_sc_params(),
    )


def _slot_count(cref, t, bucket):
    crow = cref[t, pl.ds(0, L)]
    ecnt = jnp.sum(jnp.where(_iota() == bucket, crow, 0))
    return ecnt, ecnt // CH + 1


def _spmm2_body(n_out, nbk, cap, x_hbm, rec_hbm, vals_hbm, counts_hbm,
                out_hbm, gath, edata, valv, idxv, cref, acc, sem):
    c = lax.axis_index("c")
    s = lax.axis_index("s")
    nb = nbk // NC
    zero = jnp.zeros((L,), _f32)
    pltpu.sync_copy(counts_hbm, cref)

    for b in range(nb):
        bucket = c + NC * b
        row_base = bucket * BSTR

        def zb_body(i, carry):
            for j in range(D // L):
                gath[i, pl.ds(j * L, L)] = zero
            return carry

        lax.fori_loop(0, 128, zb_body, 0)
        for k in range(6):
            pltpu.sync_copy(gath, acc.at[pl.ds(s * RSL + k * 128, 128)])
        pltpu.sync_copy(gath.at[pl.ds(0, RSL - 6 * 128)],
                        acc.at[pl.ds(s * RSL + 6 * 128, RSL - 6 * 128)])
        plsc.subcore_barrier()

        for tt in range(NC):
            t = s + NS * tt
            ecnt, nch = _slot_count(cref, t, bucket)
            rb0 = (t * nbk + bucket) * (cap * 2)
            vb0 = (t * nbk + bucket) * cap

            def chunk_body(i, carry):
                pltpu.sync_copy(
                    rec_hbm.at[pl.ds(rb0 + i * (2 * CH), 2 * CH)], edata)
                pltpu.sync_copy(vals_hbm.at[pl.ds(vb0 + i * CH, CH)], valv)
                for g in range(CH // L):
                    r16 = edata[pl.ds(g * L, L)]
                    li = r16 - row_base
                    owned = (li >= 0) & (li < BSTR)
                    idxv[pl.ds(g * L, L)] = jnp.where(owned, li, DUMMY)
                pltpu.async_copy(x_hbm.at[edata.at[pl.ds(CH, CH)]], gath,
                                 sem).wait()

                def scale_body(ts, _):
                    for u in range(8):
                        e = ts * 8 + u
                        v = plsc.load_gather(valv, [jnp.full((L,), e, _i32)])
                        for j in range(D // L):
                            sl = pl.ds(j * L, L)
                            gath[e, sl] = gath[e, sl] * v
                    return _

                lax.fori_loop(0, CH // 8, scale_body, 0)
                pltpu.sync_copy(gath, acc.at[idxv], add=True)
                return carry

            lax.fori_loop(0, nch, chunk_body, 0)
        plsc.subcore_barrier()

        short_c = (nbk - 1) % NC
        is_short_b = (b == (nbk - 1) // NC)
        tail = n_out - (nbk - 1) * BSTR - 15 * FCH

        def _full_flush():
            pltpu.sync_copy(acc.at[pl.ds(s * FCH, FCH)],
                            out_hbm.at[pl.ds(row_base + s * FCH, FCH)])

        if is_short_b:
            @pl.when((c != short_c) | (s < 15))
            def _():
                _full_flush()

            @pl.when((c == short_c) & (s == 15))
            def _():
                pltpu.sync_copy(acc.at[pl.ds(15 * FCH, tail)],
                                out_hbm.at[pl.ds(row_base + 15 * FCH, tail)])
        else:
            _full_flush()
        plsc.subcore_barrier()


@functools.lru_cache(maxsize=None)
def _make_spmm2(n_x, n_out, nbk, cap):
    scratch = [
        pltpu.VMEM((CH, D), _f32),
        pltpu.VMEM((2 * CH,), _i32),
        pltpu.VMEM((CH,), _f32),
        pltpu.VMEM((CH,), _i32),
        pltpu.VMEM((NW, L), _i32),
        pltpu.VMEM_SHARED((BACC, D), _f32),
        pltpu.SemaphoreType.DMA,
    ]
    return pl.kernel(
        functools.partial(_spmm2_body, n_out, nbk, cap),
        out_type=jax.ShapeDtypeStruct((n_out, D), _f32),
        mesh=_mesh(),
        scratch_types=scratch,
        compiler_params=_sc_params(),
    )


def _sims2_body(nbk, cap, an_hbm, rec_hbm, counts_hbm, sims_hbm, part_hbm,
                gA, gB, edata, simbuf, accv, cref, sem1, sem2):
    c = lax.axis_index("c")
    s = lax.axis_index("s")
    wid = s * NC + c
    iota = _iota()
    pltpu.sync_copy(counts_hbm, cref)
    accv[...] = jnp.zeros((L,), _f32)

    for b in range(nbk):
        ecnt, nch = _slot_count(cref, wid, b)
        rb0 = (wid * nbk + b) * (cap * 2)
        sb0 = (wid * nbk + b) * cap

        def chunk_body(i, carry):
            pltpu.sync_copy(rec_hbm.at[pl.ds(rb0 + i * (2 * CH), 2 * CH)],
                            edata)
            d1 = pltpu.async_copy(an_hbm.at[edata.at[pl.ds(0, CH)]], gA, sem1)
            d2 = pltpu.async_copy(an_hbm.at[edata.at[pl.ds(CH, CH)]], gB,
                                  sem2)
            d1.wait()
            d2.wait()
            for g in range(CH // L):
                F = jnp.zeros((L,), _f32)
                for u in range(L):
                    e = g * L + u
                    p = gA[e, pl.ds(0, L)] * gB[e, pl.ds(0, L)]
                    for j in range(1, D // L):
                        sl = pl.ds(j * L, L)
                        p = p + gA[e, sl] * gB[e, sl]
                    F = jnp.where(iota == u, jnp.sum(p), F)
                sim = (F + 1.0) * 0.5
                simbuf[pl.ds(g * L, L)] = sim
                lid = i * CH + g * L + iota
                accv[...] = accv[...] + jnp.where(lid < ecnt, sim, 0.0)
            pltpu.sync_copy(simbuf, sims_hbm.at[pl.ds(sb0 + i * CH, CH)])
            return carry

        lax.fori_loop(0, nch, chunk_body, 0)
    pltpu.sync_copy(accv, part_hbm.at[wid])


@functools.lru_cache(maxsize=None)
def _make_sims2(nbk, cap):
    scratch = [
        pltpu.VMEM((CH, D), _f32),
        pltpu.VMEM((CH, D), _f32),
        pltpu.VMEM((2 * CH,), _i32),
        pltpu.VMEM((CH,), _f32),
        pltpu.VMEM((L,), _f32),
        pltpu.VMEM((NW, L), _i32),
        pltpu.SemaphoreType.DMA,
        pltpu.SemaphoreType.DMA,
    ]
    nslot = NW * nbk
    return pl.kernel(
        functools.partial(_sims2_body, nbk, cap),
        out_type=(jax.ShapeDtypeStruct((nslot * cap,), _f32),
                  jax.ShapeDtypeStruct((NW, L), _f32)),
        mesh=_mesh(),
        scratch_types=scratch,
        compiler_params=_sc_params(),
    )


def _mean_threshold(partv):
    tot = jnp.zeros((L,), _f32)
    for t in range(NW):
        tot = tot + partv[t, pl.ds(0, L)]
    mean = jnp.sum(tot) * (1.0 / E_S)
    return jnp.where(mean > 0.7, 0.8, 0.0)


def _diags2_body(nbk, cap, sims_hbm, rec_hbm, counts_hbm, part_hbm, diags_hbm,
                 partv, bins, edata, simv, tmp, accb, cref, stage):
    c = lax.axis_index("c")
    s = lax.axis_index("s")
    iota = _iota()
    pltpu.sync_copy(counts_hbm, cref)
    pltpu.sync_copy(part_hbm, partv)
    prun = _mean_threshold(partv)
    row_base = c * BSTR

    def zero_body(i, carry):
        bins[pl.ds(i * L, L)] = jnp.zeros((L,), _f32)
        return carry

    lax.fori_loop(0, BACC // L, zero_body, 0)

    for tt in range(NC):
        t = s + NS * tt
        ecnt, nch = _slot_count(cref, t, c)
        rb0 = (t * nbk + c) * (cap * 2)
        sb0 = (t * nbk + c) * cap

        def chunk_body(i, carry):
            pltpu.sync_copy(rec_hbm.at[pl.ds(rb0 + i * (2 * CH), CH)], edata)
            pltpu.sync_copy(sims_hbm.at[pl.ds(sb0 + i * CH, CH)], simv)
            for g in range(CH // L):
                sv = simv[pl.ds(g * L, L)]
                r16 = edata[pl.ds(g * L, L)]
                lid = i * CH + g * L + iota
                pruned = jnp.where(sv < prun, 0.0, sv)
                li = r16 - row_base
                ok = (li >= 0) & (li < BSTR)
                idx = jnp.where(ok, li, DUMMY)
                val = jnp.where(ok & (lid < ecnt), pruned, 0.0)
                plsc.addupdate_scatter(bins, [idx], val)
            return carry

        lax.fori_loop(0, nch, chunk_body, 0)
    pltpu.sync_copy(bins, stage.at[s])
    plsc.subcore_barrier()

    def zero_acc(i, carry):
        accb[pl.ds(i * L, L)] = jnp.zeros((L,), _f32)
        return carry

    lax.fori_loop(0, RSL // L, zero_acc, 0)

    def red_body(t, carry):
        pltpu.sync_copy(stage.at[t, pl.ds(s * RSL, RSL)], tmp)

        def add_body(r, carry2):
            sl = pl.ds(r * L, L)
            accb[sl] = accb[sl] + tmp[sl]
            return carry2

        lax.fori_loop(0, RSL // L, add_body, 0)
        return carry

    lax.fori_loop(0, NS, red_body, 0)

    def recip_body(r, carry):
        sl = pl.ds(r * L, L)
        accb[sl] = 1.0 / (accb[sl] + 1e-7)
        return carry

    lax.fori_loop(0, RSL // L, recip_body, 0)

    @pl.when(s < 15)
    def _():
        pltpu.sync_copy(accb, diags_hbm.at[c, pl.ds(s * RSL, RSL)])

    @pl.when((s == 15) & (c == 0))
    def _():
        pltpu.sync_copy(accb.at[pl.ds(0, BSTR - 15 * RSL)],
                        diags_hbm.at[c, pl.ds(15 * RSL, BSTR - 15 * RSL)])

    @pl.when((s == 15) & (c == 1))
    def _():
        pltpu.sync_copy(accb.at[pl.ds(0, U_N - BSTR - 15 * RSL)],
                        diags_hbm.at[c, pl.ds(15 * RSL, U_N - BSTR - 15 * RSL)])


@functools.lru_cache(maxsize=None)
def _make_diags2(nbk, cap):
    scratch = [
        pltpu.VMEM((NW, L), _f32),
        pltpu.VMEM((BACC,), _f32),
        pltpu.VMEM((CH,), _i32),
        pltpu.VMEM((CH,), _f32),
        pltpu.VMEM((RSL,), _f32),
        pltpu.VMEM((RSL,), _f32),
        pltpu.VMEM((NW, L), _i32),
        pltpu.VMEM_SHARED((NS, BACC), _f32),
    ]
    return pl.kernel(
        functools.partial(_diags2_body, nbk, cap),
        out_type=jax.ShapeDtypeStruct((NC, BSTR), _f32),
        mesh=_mesh(),
        scratch_types=scratch,
        compiler_params=_sc_params(),
    )


def _nvals2_body(nbk, cap, sims_hbm, rec_hbm, counts_hbm, diags_hbm, part_hbm,
                 out_hbm, dbuf, partv, edata, simv, outv, cref):
    c = lax.axis_index("c")
    s = lax.axis_index("s")
    wid = s * NC + c
    iota = _iota()
    pltpu.sync_copy(counts_hbm, cref)
    pltpu.sync_copy(part_hbm, partv)
    prun = _mean_threshold(partv)
    pltpu.sync_copy(diags_hbm, dbuf)

    for b in range(nbk):
        ecnt, nch = _slot_count(cref, wid, b)
        rb0 = (wid * nbk + b) * (cap * 2)
        sb0 = (wid * nbk + b) * cap

        def chunk_body(i, carry):
            pltpu.sync_copy(rec_hbm.at[pl.ds(rb0 + i * (2 * CH), CH)], edata)
            pltpu.sync_copy(sims_hbm.at[pl.ds(sb0 + i * CH, CH)], simv)
            for g in range(CH // L):
                sv = simv[pl.ds(g * L, L)]
                r16 = edata[pl.ds(g * L, L)]
                lid = i * CH + g * L + iota
                pruned = jnp.where(sv < prun, 0.0, sv)
                pruned = jnp.where(lid < ecnt, pruned, 0.0)
                bkt = jnp.where(r16 >= BSTR, 1, 0)
                li = r16 - bkt * BSTR
                dg = plsc.load_gather(dbuf, [bkt, li])
                outv[pl.ds(g * L, L)] = pruned * dg
            pltpu.sync_copy(outv, out_hbm.at[pl.ds(sb0 + i * CH, CH)])
            return carry

        lax.fori_loop(0, nch, chunk_body, 0)


@functools.lru_cache(maxsize=None)
def _make_nvals2(nbk, cap):
    scratch = [
        pltpu.VMEM((NC, BSTR), _f32),
        pltpu.VMEM((NW, L), _f32),
        pltpu.VMEM((CH,), _i32),
        pltpu.VMEM((CH,), _f32),
        pltpu.VMEM((CH,), _f32),
        pltpu.VMEM((NW, L), _i32),
    ]
    nslot = NW * nbk
    return pl.kernel(
        functools.partial(_nvals2_body, nbk, cap),
        out_type=jax.ShapeDtypeStruct((nslot * cap,), _f32),
        mesh=_mesh(),
        scratch_types=scratch,
        compiler_params=_sc_params(),
    )


_RB = 1000


def _norm_body(x_ref, o_ref):
    x = x_ref[...]
    n = jnp.sqrt(jnp.sum(x * x, axis=1, keepdims=True))
    o_ref[...] = x / jnp.maximum(n, 1e-8)


def _normalize(x):
    return pl.pallas_call(
        _norm_body,
        grid=(U_N // _RB,),
        in_specs=[pl.BlockSpec((_RB, D), lambda i: (i, 0))],
        out_specs=pl.BlockSpec((_RB, D), lambda i: (i, 0)),
        out_shape=jax.ShapeDtypeStruct((U_N, D), _f32),
    )(x)


def _ego_body(u_ref, l1_ref, l2_ref, l3_ref, it_ref, ego_ref, sv_ref):
    i = pl.program_id(0)
    sv = (l1_ref[...] + l2_ref[...] + l3_ref[...]) * (1.0 / 3.0)
    sv_ref[...] = sv

    @pl.when(i < U_N // _RB)
    def _():
        ego_ref[...] = u_ref[...] + sv

    @pl.when(i >= U_N // _RB)
    def _():
        ego_ref[...] = it_ref[...]


def _ego(user_emb, l1, l2, l3, item_emb):
    nu = U_N // _RB
    user_map = lambda i: (jnp.minimum(i, nu - 1), 0)
    item_map = lambda i: (jnp.maximum(i - nu, 0), 0)
    return pl.pallas_call(
        _ego_body,
        grid=((U_N + I_N) // _RB,),
        in_specs=[
            pl.BlockSpec((_RB, D), user_map),
            pl.BlockSpec((_RB, D), user_map),
            pl.BlockSpec((_RB, D), user_map),
            pl.BlockSpec((_RB, D), user_map),
            pl.BlockSpec((_RB, D), item_map),
        ],
        out_specs=(
            pl.BlockSpec((_RB, D), lambda i: (i, 0)),
            pl.BlockSpec((_RB, D), user_map),
        ),
        out_shape=(jax.ShapeDtypeStruct((U_N + I_N, D), _f32),
                   jax.ShapeDtypeStruct((U_N, D), _f32)),
    )(user_emb, l1, l2, l3, item_emb)


def _final_body(e0u_ref, e1u_ref, e2u_ref, e0i_ref, e1i_ref, e2i_ref,
                sv_ref, g1_ref, g2_ref, user_ref, item_ref):
    uv1 = (e0u_ref[...] + e1u_ref[...] + e2u_ref[...]) * (1.0 / 3.0)
    iv1 = (e0i_ref[...] + e1i_ref[...] + e2i_ref[...]) * (1.0 / 3.0)
    item_ref[...] = iv1
    sv = sv_ref[...]
    dn = (((1,), (1,)), ((), ()))
    z = (lax.dot_general(uv1, g1_ref[...], dn, preferred_element_type=_f32)
         + lax.dot_general(sv, g2_ref[...], dn, preferred_element_type=_f32))
    g = jax.nn.sigmoid(z)
    user_ref[...] = g * sv + (1.0 - g) * uv1


def _final(ego, e1, e2, sview, g1, g2):
    nu = U_N // _RB
    umap = lambda i: (i, 0)
    imap = lambda i: (i + nu, 0)
    wmap = lambda i: (0, 0)
    return pl.pallas_call(
        _final_body,
        grid=(nu,),
        in_specs=[
            pl.BlockSpec((_RB, D), umap),
            pl.BlockSpec((_RB, D), umap),
            pl.BlockSpec((_RB, D), umap),
            pl.BlockSpec((_RB, D), imap),
            pl.BlockSpec((_RB, D), imap),
            pl.BlockSpec((_RB, D), imap),
            pl.BlockSpec((_RB, D), umap),
            pl.BlockSpec((D, D), wmap),
            pl.BlockSpec((D, D), wmap),
        ],
        out_specs=(pl.BlockSpec((_RB, D), umap),
                   pl.BlockSpec((_RB, D), umap)),
        out_shape=(jax.ShapeDtypeStruct((U_N, D), _f32),
                   jax.ShapeDtypeStruct((U_N, D), _f32)),
    )(ego, e1, e2, ego, e1, e2, sview, g1, g2)


def kernel(user_emb, item_emb, gate1_W, gate2_W, social_index, social_values,
           norm_index, norm_values):
    s_rows = jnp.pad(social_index[0], (0, ES_P - E_S))
    s_cols = jnp.pad(social_index[1], (0, ES_P - E_S))
    s_vals = jnp.pad(social_values, (0, ES_P - E_S))
    u_rows = jnp.pad(norm_index[0], (0, EU_P - E_UI))
    u_cols = jnp.pad(norm_index[1], (0, EU_P - E_UI))
    u_vals = jnp.pad(norm_values, (0, EU_P - E_UI))

    rec_s, vslot_s, cnt_s = _make_part(ES_P, E_S, NBK_S, CAP_S)(
        s_rows, s_cols, s_vals)
    rec_u, vslot_u, cnt_u = _make_part(EU_P, E_UI, NBK_UI, CAP_UI)(
        u_rows, u_cols, u_vals)

    spmm_s = _make_spmm2(U_N, U_N, NBK_S, CAP_S)
    spmm_ui = _make_spmm2(U_N + I_N, U_N + I_N, NBK_UI, CAP_UI)

    u_agg = spmm_s(user_emb, rec_s, vslot_s, cnt_s)
    a_norm = _normalize(u_agg)
    sims, parts = _make_sims2(NBK_S, CAP_S)(a_norm, rec_s, cnt_s)
    diags = _make_diags2(NBK_S, CAP_S)(sims, rec_s, cnt_s, parts)
    nvals = _make_nvals2(NBK_S, CAP_S)(sims, rec_s, cnt_s, diags, parts)
    l1 = spmm_s(user_emb, rec_s, nvals, cnt_s)
    l2 = spmm_s(l1, rec_s, nvals, cnt_s)
    l3 = spmm_s(l2, rec_s, nvals, cnt_s)
    ego, sview = _ego(user_emb, l1, l2, l3, item_emb)
    e1 = spmm_ui(ego, rec_u, vslot_u, cnt_u)
    e2 = spmm_ui(e1, rec_u, vslot_u, cnt_u)
    return _final(ego, e1, e2, sview, gate1_W, gate2_W)


# speedup vs baseline: 4.4544x; 1.2605x over previous
import functools

import jax
import jax.numpy as jnp
from jax import lax
from jax.experimental import pallas as pl
from jax.experimental.pallas import tpu as pltpu
from jax.experimental.pallas import tpu_sc as plsc

U_N = 25000
I_N = 25000
D = 128
E_S = 400000
E_UI = 600000

NC = 2
NS = 16
L = 16
CH = 96

ES_P = 402432
EU_P = 602112

BSTR = 12544
BACC = 12800
DUMMY = 12600
FCH = 784
RSL = 800

NBK_S = 2
NBK_UI = 4
NW = NC * NS
CAP_S = ES_P // NW + CH
CAP_UI = EU_P // NW + CH

_f32 = jnp.float32
_i32 = jnp.int32


def _mesh():
    return plsc.VectorSubcoreMesh(core_axis_name="c", subcore_axis_name="s")


def _sc_params():
    return pltpu.CompilerParams(needs_layout_passes=False,
                                use_tc_tiling_on_sc=False)


def _iota():
    return lax.broadcasted_iota(_i32, (L,), 0)


def _part_body(epad, ereal, nbk, cap, rows_hbm, cols_hbm, vals_hbm,
               rec_hbm, vslot_hbm, counts_hbm, *scr):
    c = lax.axis_index("c")
    s = lax.axis_index("s")
    wid = s * NC + c
    et = epad // NW
    chunks = et // CH
    iota = _iota()
    rowv, colv, valv, countv = scr[0], scr[1], scr[2], scr[3]
    st_r = scr[4:4 + nbk]
    st_c = scr[4 + nbk:4 + 2 * nbk]
    st_v = scr[4 + 2 * nbk:4 + 3 * nbk]
    ebase = wid * et

    def chunk_body(i, carry):
        fills = list(carry[0:nbk])
        curs = list(carry[nbk:2 * nbk])
        ecnt = list(carry[2 * nbk:3 * nbk])
        base = ebase + i * CH
        pltpu.sync_copy(rows_hbm.at[pl.ds(base, CH)], rowv)
        pltpu.sync_copy(cols_hbm.at[pl.ds(base, CH)], colv)
        pltpu.sync_copy(vals_hbm.at[pl.ds(base, CH)], valv)
        for g in range(CH // L):
            sl = pl.ds(g * L, L)
            r16 = rowv[sl]
            c16 = colv[sl]
            v16 = valv[sl]
            real = (base + g * L + iota) < ereal
            for b in range(nbk):
                m = real & (r16 >= b * BSTR) & (r16 < (b + 1) * BSTR)
                plsc.store_compressed(st_r[b].at[pl.ds(fills[b], L)], r16,
                                      mask=m)
                plsc.store_compressed(st_c[b].at[pl.ds(fills[b], L)], c16,
                                      mask=m)
                plsc.store_compressed(st_v[b].at[pl.ds(fills[b], L)], v16,
                                      mask=m)
                pc = jnp.max(plsc.all_reduce_population_count(m))
                fills[b] = fills[b] + pc
                ecnt[b] = ecnt[b] + pc
        for b in range(nbk):
            do = fills[b] >= CH
            rb = (wid * nbk + b) * (cap * 2)
            vb = (wid * nbk + b) * cap

            @pl.when(do)
            def _(b=b, rb=rb, vb=vb, cur=curs[b]):
                pltpu.sync_copy(st_r[b].at[pl.ds(0, CH)],
                                rec_hbm.at[pl.ds(rb + cur * (2 * CH), CH)])
                pltpu.sync_copy(st_c[b].at[pl.ds(0, CH)],
                                rec_hbm.at[pl.ds(rb + cur * (2 * CH) + CH,
                                                 CH)])
                pltpu.sync_copy(st_v[b].at[pl.ds(0, CH)],
                                vslot_hbm.at[pl.ds(vb + cur * CH, CH)])
                for j in range(CH // L):
                    lo = pl.ds(j * L, L)
                    hi = pl.ds(CH + j * L, L)
                    st_r[b][lo] = st_r[b][hi]
                    st_c[b][lo] = st_c[b][hi]
                    st_v[b][lo] = st_v[b][hi]

            fills[b] = jnp.where(do, fills[b] - CH, fills[b])
            curs[b] = jnp.where(do, curs[b] + 1, curs[b])
        return tuple(fills) + tuple(curs) + tuple(ecnt)

    z = jnp.int32(0)
    res = lax.fori_loop(0, chunks, chunk_body, (z,) * (3 * nbk))
    fills = res[0:nbk]
    curs = res[nbk:2 * nbk]
    ecnt = res[2 * nbk:3 * nbk]
    cv = jnp.zeros((L,), _i32)
    for b in range(nbk):
        for j in range(CH // L):
            sl = pl.ds(j * L, L)
            m = (j * L + iota) >= fills[b]
            st_r[b][sl] = jnp.where(m, b * BSTR, st_r[b][sl])
            st_c[b][sl] = jnp.where(m, 0, st_c[b][sl])
            st_v[b][sl] = jnp.where(m, 0.0, st_v[b][sl])
        rb = (wid * nbk + b) * (cap * 2)
        vb = (wid * nbk + b) * cap
        pltpu.sync_copy(st_r[b].at[pl.ds(0, CH)],
                        rec_hbm.at[pl.ds(rb + curs[b] * (2 * CH), CH)])
        pltpu.sync_copy(st_c[b].at[pl.ds(0, CH)],
                        rec_hbm.at[pl.ds(rb + curs[b] * (2 * CH) + CH, CH)])
        pltpu.sync_copy(st_v[b].at[pl.ds(0, CH)],
                        vslot_hbm.at[pl.ds(vb + curs[b] * CH, CH)])
        cv = jnp.where(iota == b, ecnt[b], cv)
    countv[...] = cv
    pltpu.sync_copy(countv, counts_hbm.at[wid])


@functools.lru_cache(maxsize=None)
def _make_part(epad, ereal, nbk, cap):
    scratch = ([
        pltpu.VMEM((CH,), _i32),
        pltpu.VMEM((CH,), _i32),
        pltpu.VMEM((CH,), _f32),
        pltpu.VMEM((L,), _i32),
    ] + [pltpu.VMEM((2 * CH,), _i32) for _ in range(nbk)]
      + [pltpu.VMEM((2 * CH,), _i32) for _ in range(nbk)]
      + [pltpu.VMEM((2 * CH,), _f32) for _ in range(nbk)])
    nslot = NW * nbk
    return pl.kernel(
        functools.partial(_part_body, epad, ereal, nbk, cap),
        out_type=(jax.ShapeDtypeStruct((nslot * cap * 2,), _i32),
                  jax.ShapeDtypeStruct((nslot * cap,), _f32),
                  jax.ShapeDtypeStruct((NW, L), _i32)),
        mesh=_mesh(),
        scratch_types=scratch,
        compiler_params=_sc_params(),
    )


def _slot_count(cref, t, bucket):
    crow = cref[t, pl.ds(0, L)]
    ecnt = jnp.sum(jnp.where(_iota() == bucket, crow, 0))
    return ecnt, ecnt // CH + 1


def _spmm2_body(n_out, nbk, cap, x_hbm, rec_hbm, vals_hbm, counts_hbm,
                out_hbm, gath0, gath1, ed0, ed1, valv0, valv1, idx0, idx1,
                cref, acc, gsem0, gsem1, esem0, esem1):
    c = lax.axis_index("c")
    s = lax.axis_index("s")
    nb = nbk // NC
    zero = jnp.zeros((L,), _f32)
    gaths = (gath0, gath1)
    eds = (ed0, ed1)
    valvs = (valv0, valv1)
    idxs = (idx0, idx1)
    gsems = (gsem0, gsem1)
    esems = (esem0, esem1)
    pltpu.sync_copy(counts_hbm, cref)

    for b in range(nb):
        bucket = c + NC * b
        row_base = bucket * BSTR

        def zb_body(i, carry):
            for j in range(D // L):
                gath0[i, pl.ds(j * L, L)] = zero
            return carry

        lax.fori_loop(0, CH, zb_body, 0)
        for k in range(RSL // CH):
            pltpu.sync_copy(gath0, acc.at[pl.ds(s * RSL + k * CH, CH)])
        if RSL % CH:
            pltpu.sync_copy(gath0.at[pl.ds(0, RSL % CH)],
                            acc.at[pl.ds(s * RSL + (RSL // CH) * CH,
                                         RSL % CH)])
        plsc.subcore_barrier()

        for tt in range(NC):
            t = s + NS * tt
            ecnt, nch = _slot_count(cref, t, bucket)
            rb0 = (t * nbk + bucket) * (cap * 2)
            vb0 = (t * nbk + bucket) * cap

            pltpu.sync_copy(rec_hbm.at[pl.ds(rb0, 2 * CH)], eds[0])
            pltpu.sync_copy(vals_hbm.at[pl.ds(vb0, CH)], valvs[0])
            pltpu.async_copy(x_hbm.at[eds[0].at[pl.ds(CH, CH)]], gaths[0],
                             gsems[0])

            def pair_body(kp, carry):
                for off in range(2):
                    cur = 2 * kp + off
                    p = off
                    q = 1 - off

                    @pl.when(cur < nch)
                    def _(p=p, q=q, cur=cur):
                        nxt = cur + 1
                        more = nxt < nch

                        @pl.when(more)
                        def _():
                            pltpu.async_copy(
                                rec_hbm.at[pl.ds(rb0 + nxt * (2 * CH),
                                                 2 * CH)], eds[q], esems[q])
                            pltpu.async_copy(
                                vals_hbm.at[pl.ds(vb0 + nxt * CH, CH)],
                                valvs[q], esems[q])

                        for g in range(CH // L):
                            r16 = eds[p][pl.ds(g * L, L)]
                            li = r16 - row_base
                            owned = (li >= 0) & (li < BSTR)
                            idxs[p][pl.ds(g * L, L)] = jnp.where(owned, li,
                                                                 DUMMY)
                        pltpu.make_async_copy(
                            x_hbm.at[eds[p].at[pl.ds(CH, CH)]], gaths[p],
                            gsems[p]).wait()

                        @pl.when(more)
                        def _():
                            pltpu.make_async_copy(
                                rec_hbm.at[pl.ds(rb0 + nxt * (2 * CH),
                                                 2 * CH)], eds[q],
                                esems[q]).wait()
                            pltpu.make_async_copy(
                                vals_hbm.at[pl.ds(vb0 + nxt * CH, CH)],
                                valvs[q], esems[q]).wait()
                            pltpu.async_copy(
                                x_hbm.at[eds[q].at[pl.ds(CH, CH)]], gaths[q],
                                gsems[q])

                        def scale_body(ts_, _):
                            for u in range(8):
                                e = ts_ * 8 + u
                                v = plsc.load_gather(
                                    valvs[p], [jnp.full((L,), e, _i32)])
                                for j in range(D // L):
                                    sl = pl.ds(j * L, L)
                                    gaths[p][e, sl] = gaths[p][e, sl] * v
                            return _

                        lax.fori_loop(0, CH // 8, scale_body, 0)
                        pltpu.sync_copy(gaths[p], acc.at[idxs[p]], add=True)
                return carry

            lax.fori_loop(0, (nch + 1) // 2, pair_body, 0)
        plsc.subcore_barrier()

        short_c = (nbk - 1) % NC
        is_short_b = (b == (nbk - 1) // NC)
        tail = n_out - (nbk - 1) * BSTR - 15 * FCH

        def _full_flush():
            pltpu.sync_copy(acc.at[pl.ds(s * FCH, FCH)],
                            out_hbm.at[pl.ds(row_base + s * FCH, FCH)])

        if is_short_b:
            @pl.when((c != short_c) | (s < 15))
            def _():
                _full_flush()

            @pl.when((c == short_c) & (s == 15))
            def _():
                pltpu.sync_copy(acc.at[pl.ds(15 * FCH, tail)],
                                out_hbm.at[pl.ds(row_base + 15 * FCH, tail)])
        else:
            _full_flush()
        plsc.subcore_barrier()


@functools.lru_cache(maxsize=None)
def _make_spmm2(n_x, n_out, nbk, cap):
    scratch = [
        pltpu.VMEM((CH, D), _f32),
        pltpu.VMEM((CH, D), _f32),
        pltpu.VMEM((2 * CH,), _i32),
        pltpu.VMEM((2 * CH,), _i32),
        pltpu.VMEM((CH,), _f32),
        pltpu.VMEM((CH,), _f32),
        pltpu.VMEM((CH,), _i32),
        pltpu.VMEM((CH,), _i32),
        pltpu.VMEM((NW, L), _i32),
        pltpu.VMEM_SHARED((BACC, D), _f32),
        pltpu.SemaphoreType.DMA,
        pltpu.SemaphoreType.DMA,
        pltpu.SemaphoreType.DMA,
        pltpu.SemaphoreType.DMA,
    ]
    return pl.kernel(
        functools.partial(_spmm2_body, n_out, nbk, cap),
        out_type=jax.ShapeDtypeStruct((n_out, D), _f32),
        mesh=_mesh(),
        scratch_types=scratch,
        compiler_params=_sc_params(),
    )


def _sims2_body(nbk, cap, an_hbm, rec_hbm, counts_hbm, sims_hbm, part_hbm,
                gA, gB, edata, simbuf, accv, cref, sem1, sem2):
    c = lax.axis_index("c")
    s = lax.axis_index("s")
    wid = s * NC + c
    iota = _iota()
    pltpu.sync_copy(counts_hbm, cref)
    accv[...] = jnp.zeros((L,), _f32)

    for b in range(nbk):
        ecnt, nch = _slot_count(cref, wid, b)
        rb0 = (wid * nbk + b) * (cap * 2)
        sb0 = (wid * nbk + b) * cap

        def chunk_body(i, carry):
            pltpu.sync_copy(rec_hbm.at[pl.ds(rb0 + i * (2 * CH), 2 * CH)],
                            edata)
            d1 = pltpu.async_copy(an_hbm.at[edata.at[pl.ds(0, CH)]], gA, sem1)
            d2 = pltpu.async_copy(an_hbm.at[edata.at[pl.ds(CH, CH)]], gB,
                                  sem2)
            d1.wait()
            d2.wait()
            for g in range(CH // L):
                F = jnp.zeros((L,), _f32)
                for u in range(L):
                    e = g * L + u
                    p = gA[e, pl.ds(0, L)] * gB[e, pl.ds(0, L)]
                    for j in range(1, D // L):
                        sl = pl.ds(j * L, L)
                        p = p + gA[e, sl] * gB[e, sl]
                    F = jnp.where(iota == u, jnp.sum(p), F)
                sim = (F + 1.0) * 0.5
                simbuf[pl.ds(g * L, L)] = sim
                lid = i * CH + g * L + iota
                accv[...] = accv[...] + jnp.where(lid < ecnt, sim, 0.0)
            pltpu.sync_copy(simbuf, sims_hbm.at[pl.ds(sb0 + i * CH, CH)])
            return carry

        lax.fori_loop(0, nch, chunk_body, 0)
    pltpu.sync_copy(accv, part_hbm.at[wid])


@functools.lru_cache(maxsize=None)
def _make_sims2(nbk, cap):
    scratch = [
        pltpu.VMEM((CH, D), _f32),
        pltpu.VMEM((CH, D), _f32),
        pltpu.VMEM((2 * CH,), _i32),
        pltpu.VMEM((CH,), _f32),
        pltpu.VMEM((L,), _f32),
        pltpu.VMEM((NW, L), _i32),
        pltpu.SemaphoreType.DMA,
        pltpu.SemaphoreType.DMA,
    ]
    nslot = NW * nbk
    return pl.kernel(
        functools.partial(_sims2_body, nbk, cap),
        out_type=(jax.ShapeDtypeStruct((nslot * cap,), _f32),
                  jax.ShapeDtypeStruct((NW, L), _f32)),
        mesh=_mesh(),
        scratch_types=scratch,
        compiler_params=_sc_params(),
    )


def _mean_threshold(partv):
    tot = jnp.zeros((L,), _f32)
    for t in range(NW):
        tot = tot + partv[t, pl.ds(0, L)]
    mean = jnp.sum(tot) * (1.0 / E_S)
    return jnp.where(mean > 0.7, 0.8, 0.0)


def _diags2_body(nbk, cap, sims_hbm, rec_hbm, counts_hbm, part_hbm, diags_hbm,
                 partv, bins, edata, simv, tmp, accb, cref, stage):
    c = lax.axis_index("c")
    s = lax.axis_index("s")
    iota = _iota()
    pltpu.sync_copy(counts_hbm, cref)
    pltpu.sync_copy(part_hbm, partv)
    prun = _mean_threshold(partv)
    row_base = c * BSTR

    def zero_body(i, carry):
        bins[pl.ds(i * L, L)] = jnp.zeros((L,), _f32)
        return carry

    lax.fori_loop(0, BACC // L, zero_body, 0)

    for tt in range(NC):
        t = s + NS * tt
        ecnt, nch = _slot_count(cref, t, c)
        rb0 = (t * nbk + c) * (cap * 2)
        sb0 = (t * nbk + c) * cap

        def chunk_body(i, carry):
            pltpu.sync_copy(rec_hbm.at[pl.ds(rb0 + i * (2 * CH), CH)], edata)
            pltpu.sync_copy(sims_hbm.at[pl.ds(sb0 + i * CH, CH)], simv)
            for g in range(CH // L):
                sv = simv[pl.ds(g * L, L)]
                r16 = edata[pl.ds(g * L, L)]
                lid = i * CH + g * L + iota
                pruned = jnp.where(sv < prun, 0.0, sv)
                li = r16 - row_base
                ok = (li >= 0) & (li < BSTR)
                idx = jnp.where(ok, li, DUMMY)
                val = jnp.where(ok & (lid < ecnt), pruned, 0.0)
                plsc.addupdate_scatter(bins, [idx], val)
            return carry

        lax.fori_loop(0, nch, chunk_body, 0)
    pltpu.sync_copy(bins, stage.at[s])
    plsc.subcore_barrier()

    def zero_acc(i, carry):
        accb[pl.ds(i * L, L)] = jnp.zeros((L,), _f32)
        return carry

    lax.fori_loop(0, RSL // L, zero_acc, 0)

    def red_body(t, carry):
        pltpu.sync_copy(stage.at[t, pl.ds(s * RSL, RSL)], tmp)

        def add_body(r, carry2):
            sl = pl.ds(r * L, L)
            accb[sl] = accb[sl] + tmp[sl]
            return carry2

        lax.fori_loop(0, RSL // L, add_body, 0)
        return carry

    lax.fori_loop(0, NS, red_body, 0)

    def recip_body(r, carry):
        sl = pl.ds(r * L, L)
        accb[sl] = 1.0 / (accb[sl] + 1e-7)
        return carry

    lax.fori_loop(0, RSL // L, recip_body, 0)

    @pl.when(s < 15)
    def _():
        pltpu.sync_copy(accb, diags_hbm.at[c, pl.ds(s * RSL, RSL)])

    @pl.when((s == 15) & (c == 0))
    def _():
        pltpu.sync_copy(accb.at[pl.ds(0, BSTR - 15 * RSL)],
                        diags_hbm.at[c, pl.ds(15 * RSL, BSTR - 15 * RSL)])

    @pl.when((s == 15) & (c == 1))
    def _():
        pltpu.sync_copy(accb.at[pl.ds(0, U_N - BSTR - 15 * RSL)],
                        diags_hbm.at[c, pl.ds(15 * RSL, U_N - BSTR - 15 * RSL)])


@functools.lru_cache(maxsize=None)
def _make_diags2(nbk, cap):
    scratch = [
        pltpu.VMEM((NW, L), _f32),
        pltpu.VMEM((BACC,), _f32),
        pltpu.VMEM((CH,), _i32),
        pltpu.VMEM((CH,), _f32),
        pltpu.VMEM((RSL,), _f32),
        pltpu.VMEM((RSL,), _f32),
        pltpu.VMEM((NW, L), _i32),
        pltpu.VMEM_SHARED((NS, BACC), _f32),
    ]
    return pl.kernel(
        functools.partial(_diags2_body, nbk, cap),
        out_type=jax.ShapeDtypeStruct((NC, BSTR), _f32),
        mesh=_mesh(),
        scratch_types=scratch,
        compiler_params=_sc_params(),
    )


def _nvals2_body(nbk, cap, sims_hbm, rec_hbm, counts_hbm, diags_hbm, part_hbm,
                 out_hbm, dbuf, partv, edata, simv, outv, cref):
    c = lax.axis_index("c")
    s = lax.axis_index("s")
    wid = s * NC + c
    iota = _iota()
    pltpu.sync_copy(counts_hbm, cref)
    pltpu.sync_copy(part_hbm, partv)
    prun = _mean_threshold(partv)
    pltpu.sync_copy(diags_hbm, dbuf)

    for b in range(nbk):
        ecnt, nch = _slot_count(cref, wid, b)
        rb0 = (wid * nbk + b) * (cap * 2)
        sb0 = (wid * nbk + b) * cap

        def chunk_body(i, carry):
            pltpu.sync_copy(rec_hbm.at[pl.ds(rb0 + i * (2 * CH), CH)], edata)
            pltpu.sync_copy(sims_hbm.at[pl.ds(sb0 + i * CH, CH)], simv)
            for g in range(CH // L):
                sv = simv[pl.ds(g * L, L)]
                r16 = edata[pl.ds(g * L, L)]
                lid = i * CH + g * L + iota
                pruned = jnp.where(sv < prun, 0.0, sv)
                pruned = jnp.where(lid < ecnt, pruned, 0.0)
                bkt = jnp.where(r16 >= BSTR, 1, 0)
                li = r16 - bkt * BSTR
                dg = plsc.load_gather(dbuf, [bkt, li])
                outv[pl.ds(g * L, L)] = pruned * dg
            pltpu.sync_copy(outv, out_hbm.at[pl.ds(sb0 + i * CH, CH)])
            return carry

        lax.fori_loop(0, nch, chunk_body, 0)


@functools.lru_cache(maxsize=None)
def _make_nvals2(nbk, cap):
    scratch = [
        pltpu.VMEM((NC, BSTR), _f32),
        pltpu.VMEM((NW, L), _f32),
        pltpu.VMEM((CH,), _i32),
        pltpu.VMEM((CH,), _f32),
        pltpu.VMEM((CH,), _f32),
        pltpu.VMEM((NW, L), _i32),
    ]
    nslot = NW * nbk
    return pl.kernel(
        functools.partial(_nvals2_body, nbk, cap),
        out_type=jax.ShapeDtypeStruct((nslot * cap,), _f32),
        mesh=_mesh(),
        scratch_types=scratch,
        compiler_params=_sc_params(),
    )


_RB = 1000


def _norm_body(x_ref, o_ref):
    x = x_ref[...]
    n = jnp.sqrt(jnp.sum(x * x, axis=1, keepdims=True))
    o_ref[...] = x / jnp.maximum(n, 1e-8)


def _normalize(x):
    return pl.pallas_call(
        _norm_body,
        grid=(U_N // _RB,),
        in_specs=[pl.BlockSpec((_RB, D), lambda i: (i, 0))],
        out_specs=pl.BlockSpec((_RB, D), lambda i: (i, 0)),
        out_shape=jax.ShapeDtypeStruct((U_N, D), _f32),
    )(x)


def _ego_body(u_ref, l1_ref, l2_ref, l3_ref, it_ref, ego_ref, sv_ref):
    i = pl.program_id(0)
    sv = (l1_ref[...] + l2_ref[...] + l3_ref[...]) * (1.0 / 3.0)
    sv_ref[...] = sv

    @pl.when(i < U_N // _RB)
    def _():
        ego_ref[...] = u_ref[...] + sv

    @pl.when(i >= U_N // _RB)
    def _():
        ego_ref[...] = it_ref[...]


def _ego(user_emb, l1, l2, l3, item_emb):
    nu = U_N // _RB
    user_map = lambda i: (jnp.minimum(i, nu - 1), 0)
    item_map = lambda i: (jnp.maximum(i - nu, 0), 0)
    return pl.pallas_call(
        _ego_body,
        grid=((U_N + I_N) // _RB,),
        in_specs=[
            pl.BlockSpec((_RB, D), user_map),
            pl.BlockSpec((_RB, D), user_map),
            pl.BlockSpec((_RB, D), user_map),
            pl.BlockSpec((_RB, D), user_map),
            pl.BlockSpec((_RB, D), item_map),
        ],
        out_specs=(
            pl.BlockSpec((_RB, D), lambda i: (i, 0)),
            pl.BlockSpec((_RB, D), user_map),
        ),
        out_shape=(jax.ShapeDtypeStruct((U_N + I_N, D), _f32),
                   jax.ShapeDtypeStruct((U_N, D), _f32)),
    )(user_emb, l1, l2, l3, item_emb)


def _final_body(e0u_ref, e1u_ref, e2u_ref, e0i_ref, e1i_ref, e2i_ref,
                sv_ref, g1_ref, g2_ref, user_ref, item_ref):
    uv1 = (e0u_ref[...] + e1u_ref[...] + e2u_ref[...]) * (1.0 / 3.0)
    iv1 = (e0i_ref[...] + e1i_ref[...] + e2i_ref[...]) * (1.0 / 3.0)
    item_ref[...] = iv1
    sv = sv_ref[...]
    dn = (((1,), (1,)), ((), ()))
    z = (lax.dot_general(uv1, g1_ref[...], dn, preferred_element_type=_f32)
         + lax.dot_general(sv, g2_ref[...], dn, preferred_element_type=_f32))
    g = jax.nn.sigmoid(z)
    user_ref[...] = g * sv + (1.0 - g) * uv1


def _final(ego, e1, e2, sview, g1, g2):
    nu = U_N // _RB
    umap = lambda i: (i, 0)
    imap = lambda i: (i + nu, 0)
    wmap = lambda i: (0, 0)
    return pl.pallas_call(
        _final_body,
        grid=(nu,),
        in_specs=[
            pl.BlockSpec((_RB, D), umap),
            pl.BlockSpec((_RB, D), umap),
            pl.BlockSpec((_RB, D), umap),
            pl.BlockSpec((_RB, D), imap),
            pl.BlockSpec((_RB, D), imap),
            pl.BlockSpec((_RB, D), imap),
            pl.BlockSpec((_RB, D), umap),
            pl.BlockSpec((D, D), wmap),
            pl.BlockSpec((D, D), wmap),
        ],
        out_specs=(pl.BlockSpec((_RB, D), umap),
                   pl.BlockSpec((_RB, D), umap)),
        out_shape=(jax.ShapeDtypeStruct((U_N, D), _f32),
                   jax.ShapeDtypeStruct((U_N, D), _f32)),
    )(ego, e1, e2, ego, e1, e2, sview, g1, g2)


def kernel(user_emb, item_emb, gate1_W, gate2_W, social_index, social_values,
           norm_index, norm_values):
    s_rows = jnp.pad(social_index[0], (0, ES_P - E_S))
    s_cols = jnp.pad(social_index[1], (0, ES_P - E_S))
    s_vals = jnp.pad(social_values, (0, ES_P - E_S))
    u_rows = jnp.pad(norm_index[0], (0, EU_P - E_UI))
    u_cols = jnp.pad(norm_index[1], (0, EU_P - E_UI))
    u_vals = jnp.pad(norm_values, (0, EU_P - E_UI))

    rec_s, vslot_s, cnt_s = _make_part(ES_P, E_S, NBK_S, CAP_S)(
        s_rows, s_cols, s_vals)
    rec_u, vslot_u, cnt_u = _make_part(EU_P, E_UI, NBK_UI, CAP_UI)(
        u_rows, u_cols, u_vals)

    spmm_s = _make_spmm2(U_N, U_N, NBK_S, CAP_S)
    spmm_ui = _make_spmm2(U_N + I_N, U_N + I_N, NBK_UI, CAP_UI)

    u_agg = spmm_s(user_emb, rec_s, vslot_s, cnt_s)
    a_norm = _normalize(u_agg)
    sims, parts = _make_sims2(NBK_S, CAP_S)(a_norm, rec_s, cnt_s)
    diags = _make_diags2(NBK_S, CAP_S)(sims, rec_s, cnt_s, parts)
    nvals = _make_nvals2(NBK_S, CAP_S)(sims, rec_s, cnt_s, diags, parts)
    l1 = spmm_s(user_emb, rec_s, nvals, cnt_s)
    l2 = spmm_s(l1, rec_s, nvals, cnt_s)
    l3 = spmm_s(l2, rec_s, nvals, cnt_s)
    ego, sview = _ego(user_emb, l1, l2, l3, item_emb)
    e1 = spmm_ui(ego, rec_u, vslot_u, cnt_u)
    e2 = spmm_ui(e1, rec_u, vslot_u, cnt_u)
    return _final(ego, e1, e2, sview, gate1_W, gate2_W)


# speedup vs baseline: 4.5246x; 1.0158x over previous
import functools

import jax
import jax.numpy as jnp
from jax import lax
from jax.experimental import pallas as pl
from jax.experimental.pallas import tpu as pltpu
from jax.experimental.pallas import tpu_sc as plsc

U_N = 25000
I_N = 25000
D = 128
E_S = 400000
E_UI = 600000

NC = 2
NS = 16
L = 16
CH = 96

ES_P = 402432
EU_P = 602112

BSTR = 12544
BACC = 12800
DUMMY = 12600
FCH = 784
RSL = 800

NBK_S = 2
NBK_UI = 4
NW = NC * NS
CAP_S = ES_P // NW + CH
CAP_UI = EU_P // NW + CH

_f32 = jnp.float32
_i32 = jnp.int32


def _mesh():
    return plsc.VectorSubcoreMesh(core_axis_name="c", subcore_axis_name="s")


def _sc_params():
    return pltpu.CompilerParams(needs_layout_passes=False,
                                use_tc_tiling_on_sc=False)


def _iota():
    return lax.broadcasted_iota(_i32, (L,), 0)


def _part_body(epad, ereal, nbk, cap, rows_hbm, cols_hbm, vals_hbm,
               rec_hbm, vslot_hbm, counts_hbm, *scr):
    c = lax.axis_index("c")
    s = lax.axis_index("s")
    wid = s * NC + c
    et = epad // NW
    chunks = et // CH
    iota = _iota()
    rowv, colv, valv, countv = scr[0], scr[1], scr[2], scr[3]
    st_p = scr[4:4 + nbk]
    st_v = scr[4 + nbk:4 + 2 * nbk]
    ebase = wid * et

    def chunk_body(i, carry):
        fills = list(carry[0:nbk])
        curs = list(carry[nbk:2 * nbk])
        ecnt = list(carry[2 * nbk:3 * nbk])
        base = ebase + i * CH
        pltpu.sync_copy(rows_hbm.at[pl.ds(base, CH)], rowv)
        pltpu.sync_copy(cols_hbm.at[pl.ds(base, CH)], colv)
        pltpu.sync_copy(vals_hbm.at[pl.ds(base, CH)], valv)
        pgs, vs, ms, pcs = [], [], [], []
        for g in range(CH // L):
            sl = pl.ds(g * L, L)
            r16 = rowv[sl]
            c16 = colv[sl]
            pgs.append(jnp.left_shift(r16, 17) | c16)
            vs.append(valv[sl])
            real = (base + g * L + iota) < ereal
            for b in range(nbk):
                m = real & (r16 >= b * BSTR) & (r16 < (b + 1) * BSTR)
                ms.append(m)
                pcs.append(jnp.max(plsc.all_reduce_population_count(m)))
        for g in range(CH // L):
            for b in range(nbk):
                k = g * nbk + b
                off32 = ((b * BSTR << 17) + 2**31) % 2**32 - 2**31
                packed = pgs[g] - jnp.int32(off32)
                plsc.store_compressed(st_p[b].at[pl.ds(fills[b], L)], packed,
                                      mask=ms[k])
                plsc.store_compressed(st_v[b].at[pl.ds(fills[b], L)], vs[g],
                                      mask=ms[k])
                fills[b] = fills[b] + pcs[k]
                ecnt[b] = ecnt[b] + pcs[k]
        for b in range(nbk):
            do = fills[b] >= CH
            rb = (wid * nbk + b) * cap

            @pl.when(do)
            def _(b=b, rb=rb, cur=curs[b]):
                pltpu.sync_copy(st_p[b].at[pl.ds(0, CH)],
                                rec_hbm.at[pl.ds(rb + cur * CH, CH)])
                pltpu.sync_copy(st_v[b].at[pl.ds(0, CH)],
                                vslot_hbm.at[pl.ds(rb + cur * CH, CH)])
                for j in range(CH // L):
                    lo = pl.ds(j * L, L)
                    hi = pl.ds(CH + j * L, L)
                    st_p[b][lo] = st_p[b][hi]
                    st_v[b][lo] = st_v[b][hi]

            fills[b] = jnp.where(do, fills[b] - CH, fills[b])
            curs[b] = jnp.where(do, curs[b] + 1, curs[b])
        return tuple(fills) + tuple(curs) + tuple(ecnt)

    z = jnp.int32(0)
    res = lax.fori_loop(0, chunks, chunk_body, (z,) * (3 * nbk))
    fills = res[0:nbk]
    curs = res[nbk:2 * nbk]
    ecnt = res[2 * nbk:3 * nbk]
    cv = jnp.zeros((L,), _i32)
    for b in range(nbk):
        for j in range(CH // L):
            sl = pl.ds(j * L, L)
            m = (j * L + iota) >= fills[b]
            st_p[b][sl] = jnp.where(m, 0, st_p[b][sl])
            st_v[b][sl] = jnp.where(m, 0.0, st_v[b][sl])
        rb = (wid * nbk + b) * cap
        pltpu.sync_copy(st_p[b].at[pl.ds(0, CH)],
                        rec_hbm.at[pl.ds(rb + curs[b] * CH, CH)])
        pltpu.sync_copy(st_v[b].at[pl.ds(0, CH)],
                        vslot_hbm.at[pl.ds(rb + curs[b] * CH, CH)])
        cv = jnp.where(iota == b, ecnt[b], cv)
    countv[...] = cv
    pltpu.sync_copy(countv, counts_hbm.at[wid])


@functools.lru_cache(maxsize=None)
def _make_part(epad, ereal, nbk, cap):
    scratch = ([
        pltpu.VMEM((CH,), _i32),
        pltpu.VMEM((CH,), _i32),
        pltpu.VMEM((CH,), _f32),
        pltpu.VMEM((L,), _i32),
    ] + [pltpu.VMEM((2 * CH,), _i32) for _ in range(nbk)]
      + [pltpu.VMEM((2 * CH,), _f32) for _ in range(nbk)])
    nslot = NW * nbk
    return pl.kernel(
        functools.partial(_part_body, epad, ereal, nbk, cap),
        out_type=(jax.ShapeDtypeStruct((nslot * cap,), _i32),
                  jax.ShapeDtypeStruct((nslot * cap,), _f32),
                  jax.ShapeDtypeStruct((NW, L), _i32)),
        mesh=_mesh(),
        scratch_types=scratch,
        compiler_params=_sc_params(),
    )


def _slot_count(cref, t, bucket):
    crow = cref[t, pl.ds(0, L)]
    ecnt = jnp.sum(jnp.where(_iota() == bucket, crow, 0))
    return ecnt, ecnt // CH + 1


def _spmm2_body(n_out, nbk, cap, x_hbm, rec_hbm, vals_hbm, counts_hbm,
                out_hbm, gath0, gath1, ed0, ed1, cv0, cv1, valv0, valv1,
                idx0, idx1, cref, acc, gsem0, gsem1, esem0, esem1):
    c = lax.axis_index("c")
    s = lax.axis_index("s")
    nb = nbk // NC
    zero = jnp.zeros((L,), _f32)
    gaths = (gath0, gath1)
    eds = (ed0, ed1)
    cvs = (cv0, cv1)
    valvs = (valv0, valv1)
    idxs = (idx0, idx1)
    gsems = (gsem0, gsem1)
    esems = (esem0, esem1)
    pltpu.sync_copy(counts_hbm, cref)

    def unpack_cols(p):
        for g in range(CH // L):
            sl = pl.ds(g * L, L)
            cvs[p][sl] = eds[p][sl] & 131071

    for b in range(nb):
        bucket = c + NC * b
        row_base = bucket * BSTR

        def zb_body(i, carry):
            for j in range(D // L):
                gath0[i, pl.ds(j * L, L)] = zero
            return carry

        lax.fori_loop(0, CH, zb_body, 0)
        for k in range(RSL // CH):
            pltpu.sync_copy(gath0, acc.at[pl.ds(s * RSL + k * CH, CH)])
        if RSL % CH:
            pltpu.sync_copy(gath0.at[pl.ds(0, RSL % CH)],
                            acc.at[pl.ds(s * RSL + (RSL // CH) * CH,
                                         RSL % CH)])
        plsc.subcore_barrier()

        for tt in range(NC):
            t = s + NS * tt
            ecnt, nch = _slot_count(cref, t, bucket)
            rb0 = (t * nbk + bucket) * cap

            pltpu.sync_copy(rec_hbm.at[pl.ds(rb0, CH)], eds[0])
            pltpu.sync_copy(vals_hbm.at[pl.ds(rb0, CH)], valvs[0])
            unpack_cols(0)
            pltpu.async_copy(x_hbm.at[cvs[0]], gaths[0], gsems[0])

            def pair_body(kp, carry):
                for off in range(2):
                    cur = 2 * kp + off
                    p = off
                    q = 1 - off

                    @pl.when(cur < nch)
                    def _(p=p, q=q, cur=cur):
                        nxt = cur + 1
                        more = nxt < nch

                        @pl.when(more)
                        def _():
                            pltpu.async_copy(
                                rec_hbm.at[pl.ds(rb0 + nxt * CH, CH)],
                                eds[q], esems[q])
                            pltpu.async_copy(
                                vals_hbm.at[pl.ds(rb0 + nxt * CH, CH)],
                                valvs[q], esems[q])

                        for g in range(CH // L):
                            sl = pl.ds(g * L, L)
                            idxs[p][sl] = lax.shift_right_logical(eds[p][sl],
                                                                  17)
                        pltpu.make_async_copy(x_hbm.at[cvs[p]], gaths[p],
                                              gsems[p]).wait()

                        @pl.when(more)
                        def _():
                            pltpu.make_async_copy(
                                rec_hbm.at[pl.ds(rb0 + nxt * CH, CH)],
                                eds[q], esems[q]).wait()
                            pltpu.make_async_copy(
                                vals_hbm.at[pl.ds(rb0 + nxt * CH, CH)],
                                valvs[q], esems[q]).wait()
                            unpack_cols(q)
                            pltpu.async_copy(x_hbm.at[cvs[q]], gaths[q],
                                             gsems[q])

                        def scale_body(ts_, _):
                            for u in range(8):
                                e = ts_ * 8 + u
                                v = plsc.load_gather(
                                    valvs[p], [jnp.full((L,), e, _i32)])
                                for j in range(D // L):
                                    sl = pl.ds(j * L, L)
                                    gaths[p][e, sl] = gaths[p][e, sl] * v
                            return _

                        lax.fori_loop(0, CH // 8, scale_body, 0)
                        pltpu.sync_copy(gaths[p], acc.at[idxs[p]], add=True)
                return carry

            lax.fori_loop(0, (nch + 1) // 2, pair_body, 0)
        plsc.subcore_barrier()

        short_c = (nbk - 1) % NC
        is_short_b = (b == (nbk - 1) // NC)
        tail = n_out - (nbk - 1) * BSTR - 15 * FCH

        def _full_flush():
            pltpu.sync_copy(acc.at[pl.ds(s * FCH, FCH)],
                            out_hbm.at[pl.ds(row_base + s * FCH, FCH)])

        if is_short_b:
            @pl.when((c != short_c) | (s < 15))
            def _():
                _full_flush()

            @pl.when((c == short_c) & (s == 15))
            def _():
                pltpu.sync_copy(acc.at[pl.ds(15 * FCH, tail)],
                                out_hbm.at[pl.ds(row_base + 15 * FCH, tail)])
        else:
            _full_flush()
        plsc.subcore_barrier()


@functools.lru_cache(maxsize=None)
def _make_spmm2(n_x, n_out, nbk, cap):
    scratch = [
        pltpu.VMEM((CH, D), _f32),
        pltpu.VMEM((CH, D), _f32),
        pltpu.VMEM((CH,), _i32),
        pltpu.VMEM((CH,), _i32),
        pltpu.VMEM((CH,), _i32),
        pltpu.VMEM((CH,), _i32),
        pltpu.VMEM((CH,), _f32),
        pltpu.VMEM((CH,), _f32),
        pltpu.VMEM((CH,), _i32),
        pltpu.VMEM((CH,), _i32),
        pltpu.VMEM((NW, L), _i32),
        pltpu.VMEM_SHARED((BACC, D), _f32),
        pltpu.SemaphoreType.DMA,
        pltpu.SemaphoreType.DMA,
        pltpu.SemaphoreType.DMA,
        pltpu.SemaphoreType.DMA,
    ]
    return pl.kernel(
        functools.partial(_spmm2_body, n_out, nbk, cap),
        out_type=jax.ShapeDtypeStruct((n_out, D), _f32),
        mesh=_mesh(),
        scratch_types=scratch,
        compiler_params=_sc_params(),
    )


def _sims2_body(nbk, cap, an_hbm, rec_hbm, counts_hbm, sims_hbm, part_hbm,
                gA, gB, edata, rowb, colb, simbuf, accv, cref, sem1, sem2):
    c = lax.axis_index("c")
    s = lax.axis_index("s")
    wid = s * NC + c
    iota = _iota()
    pltpu.sync_copy(counts_hbm, cref)
    accv[...] = jnp.zeros((L,), _f32)

    for b in range(nbk):
        ecnt, nch = _slot_count(cref, wid, b)
        rb0 = (wid * nbk + b) * cap

        def chunk_body(i, carry):
            pltpu.sync_copy(rec_hbm.at[pl.ds(rb0 + i * CH, CH)], edata)
            for g in range(CH // L):
                sl = pl.ds(g * L, L)
                pk = edata[sl]
                rowb[sl] = lax.shift_right_logical(pk, 17) + b * BSTR
                colb[sl] = pk & 131071
            d1 = pltpu.async_copy(an_hbm.at[rowb], gA, sem1)
            d2 = pltpu.async_copy(an_hbm.at[colb], gB, sem2)
            d1.wait()
            d2.wait()
            for g in range(CH // L):
                F = jnp.zeros((L,), _f32)
                for u in range(L):
                    e = g * L + u
                    p = gA[e, pl.ds(0, L)] * gB[e, pl.ds(0, L)]
                    for j in range(1, D // L):
                        sl = pl.ds(j * L, L)
                        p = p + gA[e, sl] * gB[e, sl]
                    F = jnp.where(iota == u, jnp.sum(p), F)
                sim = (F + 1.0) * 0.5
                simbuf[pl.ds(g * L, L)] = sim
                lid = i * CH + g * L + iota
                accv[...] = accv[...] + jnp.where(lid < ecnt, sim, 0.0)
            pltpu.sync_copy(simbuf, sims_hbm.at[pl.ds(rb0 + i * CH, CH)])
            return carry

        lax.fori_loop(0, nch, chunk_body, 0)
    pltpu.sync_copy(accv, part_hbm.at[wid])


@functools.lru_cache(maxsize=None)
def _make_sims2(nbk, cap):
    scratch = [
        pltpu.VMEM((CH, D), _f32),
        pltpu.VMEM((CH, D), _f32),
        pltpu.VMEM((CH,), _i32),
        pltpu.VMEM((CH,), _i32),
        pltpu.VMEM((CH,), _i32),
        pltpu.VMEM((CH,), _f32),
        pltpu.VMEM((L,), _f32),
        pltpu.VMEM((NW, L), _i32),
        pltpu.SemaphoreType.DMA,
        pltpu.SemaphoreType.DMA,
    ]
    nslot = NW * nbk
    return pl.kernel(
        functools.partial(_sims2_body, nbk, cap),
        out_type=(jax.ShapeDtypeStruct((nslot * cap,), _f32),
                  jax.ShapeDtypeStruct((NW, L), _f32)),
        mesh=_mesh(),
        scratch_types=scratch,
        compiler_params=_sc_params(),
    )


def _mean_threshold(partv):
    tot = jnp.zeros((L,), _f32)
    for t in range(NW):
        tot = tot + partv[t, pl.ds(0, L)]
    mean = jnp.sum(tot) * (1.0 / E_S)
    return jnp.where(mean > 0.7, 0.8, 0.0)


def _diags2_body(nbk, cap, sims_hbm, rec_hbm, counts_hbm, part_hbm, diags_hbm,
                 partv, bins, edata, simv, tmp, accb, cref, stage):
    c = lax.axis_index("c")
    s = lax.axis_index("s")
    iota = _iota()
    pltpu.sync_copy(counts_hbm, cref)
    pltpu.sync_copy(part_hbm, partv)
    prun = _mean_threshold(partv)

    def zero_body(i, carry):
        bins[pl.ds(i * L, L)] = jnp.zeros((L,), _f32)
        return carry

    lax.fori_loop(0, BACC // L, zero_body, 0)

    for tt in range(NC):
        t = s + NS * tt
        ecnt, nch = _slot_count(cref, t, c)
        rb0 = (t * nbk + c) * cap

        def chunk_body(i, carry):
            pltpu.sync_copy(rec_hbm.at[pl.ds(rb0 + i * CH, CH)], edata)
            pltpu.sync_copy(sims_hbm.at[pl.ds(rb0 + i * CH, CH)], simv)
            for g in range(CH // L):
                sv = simv[pl.ds(g * L, L)]
                idx = lax.shift_right_logical(edata[pl.ds(g * L, L)], 17)
                lid = i * CH + g * L + iota
                pruned = jnp.where(sv < prun, 0.0, sv)
                val = jnp.where(lid < ecnt, pruned, 0.0)
                plsc.addupdate_scatter(bins, [idx], val)
            return carry

        lax.fori_loop(0, nch, chunk_body, 0)
    pltpu.sync_copy(bins, stage.at[s])
    plsc.subcore_barrier()

    def zero_acc(i, carry):
        accb[pl.ds(i * L, L)] = jnp.zeros((L,), _f32)
        return carry

    lax.fori_loop(0, RSL // L, zero_acc, 0)

    def red_body(t, carry):
        pltpu.sync_copy(stage.at[t, pl.ds(s * RSL, RSL)], tmp)

        def add_body(r, carry2):
            sl = pl.ds(r * L, L)
            accb[sl] = accb[sl] + tmp[sl]
            return carry2

        lax.fori_loop(0, RSL // L, add_body, 0)
        return carry

    lax.fori_loop(0, NS, red_body, 0)

    def recip_body(r, carry):
        sl = pl.ds(r * L, L)
        accb[sl] = 1.0 / (accb[sl] + 1e-7)
        return carry

    lax.fori_loop(0, RSL // L, recip_body, 0)

    @pl.when(s < 15)
    def _():
        pltpu.sync_copy(accb, diags_hbm.at[c, pl.ds(s * RSL, RSL)])

    @pl.when((s == 15) & (c == 0))
    def _():
        pltpu.sync_copy(accb.at[pl.ds(0, BSTR - 15 * RSL)],
                        diags_hbm.at[c, pl.ds(15 * RSL, BSTR - 15 * RSL)])

    @pl.when((s == 15) & (c == 1))
    def _():
        pltpu.sync_copy(accb.at[pl.ds(0, U_N - BSTR - 15 * RSL)],
                        diags_hbm.at[c, pl.ds(15 * RSL, U_N - BSTR - 15 * RSL)])


@functools.lru_cache(maxsize=None)
def _make_diags2(nbk, cap):
    scratch = [
        pltpu.VMEM((NW, L), _f32),
        pltpu.VMEM((BACC,), _f32),
        pltpu.VMEM((CH,), _i32),
        pltpu.VMEM((CH,), _f32),
        pltpu.VMEM((RSL,), _f32),
        pltpu.VMEM((RSL,), _f32),
        pltpu.VMEM((NW, L), _i32),
        pltpu.VMEM_SHARED((NS, BACC), _f32),
    ]
    return pl.kernel(
        functools.partial(_diags2_body, nbk, cap),
        out_type=jax.ShapeDtypeStruct((NC, BSTR), _f32),
        mesh=_mesh(),
        scratch_types=scratch,
        compiler_params=_sc_params(),
    )


def _nvals2_body(nbk, cap, sims_hbm, rec_hbm, counts_hbm, diags_hbm, part_hbm,
                 out_hbm, dbuf, partv, edata, simv, outv, cref):
    c = lax.axis_index("c")
    s = lax.axis_index("s")
    wid = s * NC + c
    iota = _iota()
    pltpu.sync_copy(counts_hbm, cref)
    pltpu.sync_copy(part_hbm, partv)
    prun = _mean_threshold(partv)
    pltpu.sync_copy(diags_hbm, dbuf)

    for b in range(nbk):
        ecnt, nch = _slot_count(cref, wid, b)
        rb0 = (wid * nbk + b) * cap
        bsplat = jnp.full((L,), b, _i32)

        def chunk_body(i, carry):
            pltpu.sync_copy(rec_hbm.at[pl.ds(rb0 + i * CH, CH)], edata)
            pltpu.sync_copy(sims_hbm.at[pl.ds(rb0 + i * CH, CH)], simv)
            for g in range(CH // L):
                sv = simv[pl.ds(g * L, L)]
                li = lax.shift_right_logical(edata[pl.ds(g * L, L)], 17)
                lid = i * CH + g * L + iota
                pruned = jnp.where(sv < prun, 0.0, sv)
                pruned = jnp.where(lid < ecnt, pruned, 0.0)
                dg = plsc.load_gather(dbuf, [bsplat, li])
                outv[pl.ds(g * L, L)] = pruned * dg
            pltpu.sync_copy(outv, out_hbm.at[pl.ds(rb0 + i * CH, CH)])
            return carry

        lax.fori_loop(0, nch, chunk_body, 0)


@functools.lru_cache(maxsize=None)
def _make_nvals2(nbk, cap):
    scratch = [
        pltpu.VMEM((NC, BSTR), _f32),
        pltpu.VMEM((NW, L), _f32),
        pltpu.VMEM((CH,), _i32),
        pltpu.VMEM((CH,), _f32),
        pltpu.VMEM((CH,), _f32),
        pltpu.VMEM((NW, L), _i32),
    ]
    nslot = NW * nbk
    return pl.kernel(
        functools.partial(_nvals2_body, nbk, cap),
        out_type=jax.ShapeDtypeStruct((nslot * cap,), _f32),
        mesh=_mesh(),
        scratch_types=scratch,
        compiler_params=_sc_params(),
    )


_RB = 1000


def _norm_body(x_ref, o_ref):
    x = x_ref[...]
    n = jnp.sqrt(jnp.sum(x * x, axis=1, keepdims=True))
    o_ref[...] = x / jnp.maximum(n, 1e-8)


def _normalize(x):
    return pl.pallas_call(
        _norm_body,
        grid=(U_N // _RB,),
        in_specs=[pl.BlockSpec((_RB, D), lambda i: (i, 0))],
        out_specs=pl.BlockSpec((_RB, D), lambda i: (i, 0)),
        out_shape=jax.ShapeDtypeStruct((U_N, D), _f32),
    )(x)


def _ego_body(u_ref, l1_ref, l2_ref, l3_ref, it_ref, ego_ref, sv_ref):
    i = pl.program_id(0)
    sv = (l1_ref[...] + l2_ref[...] + l3_ref[...]) * (1.0 / 3.0)
    sv_ref[...] = sv

    @pl.when(i < U_N // _RB)
    def _():
        ego_ref[...] = u_ref[...] + sv

    @pl.when(i >= U_N // _RB)
    def _():
        ego_ref[...] = it_ref[...]


def _ego(user_emb, l1, l2, l3, item_emb):
    nu = U_N // _RB
    user_map = lambda i: (jnp.minimum(i, nu - 1), 0)
    item_map = lambda i: (jnp.maximum(i - nu, 0), 0)
    return pl.pallas_call(
        _ego_body,
        grid=((U_N + I_N) // _RB,),
        in_specs=[
            pl.BlockSpec((_RB, D), user_map),
            pl.BlockSpec((_RB, D), user_map),
            pl.BlockSpec((_RB, D), user_map),
            pl.BlockSpec((_RB, D), user_map),
            pl.BlockSpec((_RB, D), item_map),
        ],
        out_specs=(
            pl.BlockSpec((_RB, D), lambda i: (i, 0)),
            pl.BlockSpec((_RB, D), user_map),
        ),
        out_shape=(jax.ShapeDtypeStruct((U_N + I_N, D), _f32),
                   jax.ShapeDtypeStruct((U_N, D), _f32)),
    )(user_emb, l1, l2, l3, item_emb)


def _final_body(e0u_ref, e1u_ref, e2u_ref, e0i_ref, e1i_ref, e2i_ref,
                sv_ref, g1_ref, g2_ref, user_ref, item_ref):
    uv1 = (e0u_ref[...] + e1u_ref[...] + e2u_ref[...]) * (1.0 / 3.0)
    iv1 = (e0i_ref[...] + e1i_ref[...] + e2i_ref[...]) * (1.0 / 3.0)
    item_ref[...] = iv1
    sv = sv_ref[...]
    dn = (((1,), (1,)), ((), ()))
    z = (lax.dot_general(uv1, g1_ref[...], dn, preferred_element_type=_f32)
         + lax.dot_general(sv, g2_ref[...], dn, preferred_element_type=_f32))
    g = jax.nn.sigmoid(z)
    user_ref[...] = g * sv + (1.0 - g) * uv1


def _final(ego, e1, e2, sview, g1, g2):
    nu = U_N // _RB
    umap = lambda i: (i, 0)
    imap = lambda i: (i + nu, 0)
    wmap = lambda i: (0, 0)
    return pl.pallas_call(
        _final_body,
        grid=(nu,),
        in_specs=[
            pl.BlockSpec((_RB, D), umap),
            pl.BlockSpec((_RB, D), umap),
            pl.BlockSpec((_RB, D), umap),
            pl.BlockSpec((_RB, D), imap),
            pl.BlockSpec((_RB, D), imap),
            pl.BlockSpec((_RB, D), imap),
            pl.BlockSpec((_RB, D), umap),
            pl.BlockSpec((D, D), wmap),
            pl.BlockSpec((D, D), wmap),
        ],
        out_specs=(pl.BlockSpec((_RB, D), umap),
                   pl.BlockSpec((_RB, D), umap)),
        out_shape=(jax.ShapeDtypeStruct((U_N, D), _f32),
                   jax.ShapeDtypeStruct((U_N, D), _f32)),
    )(ego, e1, e2, ego, e1, e2, sview, g1, g2)


def kernel(user_emb, item_emb, gate1_W, gate2_W, social_index, social_values,
           norm_index, norm_values):
    s_rows = jnp.pad(social_index[0], (0, ES_P - E_S))
    s_cols = jnp.pad(social_index[1], (0, ES_P - E_S))
    s_vals = jnp.pad(social_values, (0, ES_P - E_S))
    u_rows = jnp.pad(norm_index[0], (0, EU_P - E_UI))
    u_cols = jnp.pad(norm_index[1], (0, EU_P - E_UI))
    u_vals = jnp.pad(norm_values, (0, EU_P - E_UI))

    rec_s, vslot_s, cnt_s = _make_part(ES_P, E_S, NBK_S, CAP_S)(
        s_rows, s_cols, s_vals)
    rec_u, vslot_u, cnt_u = _make_part(EU_P, E_UI, NBK_UI, CAP_UI)(
        u_rows, u_cols, u_vals)

    spmm_s = _make_spmm2(U_N, U_N, NBK_S, CAP_S)
    spmm_ui = _make_spmm2(U_N + I_N, U_N + I_N, NBK_UI, CAP_UI)

    u_agg = spmm_s(user_emb, rec_s, vslot_s, cnt_s)
    a_norm = _normalize(u_agg)
    sims, parts = _make_sims2(NBK_S, CAP_S)(a_norm, rec_s, cnt_s)
    diags = _make_diags2(NBK_S, CAP_S)(sims, rec_s, cnt_s, parts)
    nvals = _make_nvals2(NBK_S, CAP_S)(sims, rec_s, cnt_s, diags, parts)
    l1 = spmm_s(user_emb, rec_s, nvals, cnt_s)
    l2 = spmm_s(l1, rec_s, nvals, cnt_s)
    l3 = spmm_s(l2, rec_s, nvals, cnt_s)
    ego, sview = _ego(user_emb, l1, l2, l3, item_emb)
    e1 = spmm_ui(ego, rec_u, vslot_u, cnt_u)
    e2 = spmm_ui(e1, rec_u, vslot_u, cnt_u)
    return _final(ego, e1, e2, sview, gate1_W, gate2_W)


# speedup vs baseline: 4.8914x; 1.0811x over previous
import functools

import jax
import jax.numpy as jnp
from jax import lax
from jax.experimental import pallas as pl
from jax.experimental.pallas import tpu as pltpu
from jax.experimental.pallas import tpu_sc as plsc

U_N = 25000
I_N = 25000
D = 128
E_S = 400000
E_UI = 600000

NC = 2
NS = 16
L = 16
CH = 96

ES_P = 405504
EU_P = 602112

BSTR = 12544
BACC = 12800
DUMMY = 12600
FCH = 784
RSL = 800

NBK_S = 2
NBK_UI = 4
NW = NC * NS
CAP_S = ES_P // NW + 2 * CH
CAP_UI = EU_P // NW + 2 * CH

_f32 = jnp.float32
_i32 = jnp.int32


def _mesh():
    return plsc.VectorSubcoreMesh(core_axis_name="c", subcore_axis_name="s")


def _sc_params():
    return pltpu.CompilerParams(needs_layout_passes=False,
                                use_tc_tiling_on_sc=False)


def _iota():
    return lax.broadcasted_iota(_i32, (L,), 0)


def _part_body(epad, ereal, nbk, cap, rows_hbm, cols_hbm, vals_hbm,
               rec_hbm, vslot_hbm, counts_hbm, *scr):
    c = lax.axis_index("c")
    s = lax.axis_index("s")
    wid = s * NC + c
    et = epad // NW
    CH2 = 2 * CH
    chunks = et // CH2
    iota = _iota()
    rowv, colv, valv, countv, psem = scr[0], scr[1], scr[2], scr[3], scr[4]
    st_p = scr[5:5 + nbk]
    st_v = scr[5 + nbk:5 + 2 * nbk]
    ebase = wid * et

    def chunk_body(i, carry):
        fills = list(carry[0:nbk])
        curs = list(carry[nbk:2 * nbk])
        ecnt = list(carry[2 * nbk:3 * nbk])
        base = ebase + i * CH2
        pltpu.async_copy(rows_hbm.at[pl.ds(base, CH2)], rowv, psem)
        pltpu.async_copy(cols_hbm.at[pl.ds(base, CH2)], colv, psem)
        pltpu.async_copy(vals_hbm.at[pl.ds(base, CH2)], valv, psem)
        pltpu.make_async_copy(rows_hbm.at[pl.ds(base, CH2)], rowv,
                              psem).wait()
        pltpu.make_async_copy(cols_hbm.at[pl.ds(base, CH2)], colv,
                              psem).wait()
        pltpu.make_async_copy(vals_hbm.at[pl.ds(base, CH2)], valv,
                              psem).wait()
        pgs, vs, ms, pcs = [], [], [], []
        for g in range(CH2 // L):
            sl = pl.ds(g * L, L)
            r16 = rowv[sl]
            c16 = colv[sl]
            pgs.append(jnp.left_shift(r16, 17) | c16)
            vs.append(valv[sl])
            real = (base + g * L + iota) < ereal
            for b in range(nbk):
                m = real & (r16 >= b * BSTR) & (r16 < (b + 1) * BSTR)
                ms.append(m)
                pcs.append(jnp.max(plsc.all_reduce_population_count(m)))
        for g in range(CH2 // L):
            for b in range(nbk):
                k = g * nbk + b
                off32 = ((b * BSTR << 17) + 2**31) % 2**32 - 2**31
                packed = pgs[g] - jnp.int32(off32)
                plsc.store_compressed(st_p[b].at[pl.ds(fills[b], L)], packed,
                                      mask=ms[k])
                plsc.store_compressed(st_v[b].at[pl.ds(fills[b], L)], vs[g],
                                      mask=ms[k])
                fills[b] = fills[b] + pcs[k]
                ecnt[b] = ecnt[b] + pcs[k]
        for b in range(nbk):
            do = fills[b] >= CH2
            rb = (wid * nbk + b) * cap

            @pl.when(do)
            def _(b=b, rb=rb, cur=curs[b]):
                pltpu.sync_copy(st_p[b].at[pl.ds(0, CH2)],
                                rec_hbm.at[pl.ds(rb + cur * CH, CH2)])
                pltpu.sync_copy(st_v[b].at[pl.ds(0, CH2)],
                                vslot_hbm.at[pl.ds(rb + cur * CH, CH2)])
                for j in range(CH2 // L):
                    lo = pl.ds(j * L, L)
                    hi = pl.ds(CH2 + j * L, L)
                    st_p[b][lo] = st_p[b][hi]
                    st_v[b][lo] = st_v[b][hi]

            fills[b] = jnp.where(do, fills[b] - CH2, fills[b])
            curs[b] = jnp.where(do, curs[b] + 2, curs[b])
        return tuple(fills) + tuple(curs) + tuple(ecnt)

    z = jnp.int32(0)
    res = lax.fori_loop(0, chunks, chunk_body, (z,) * (3 * nbk))
    fills = res[0:nbk]
    curs = res[nbk:2 * nbk]
    ecnt = res[2 * nbk:3 * nbk]
    CH2 = 2 * CH
    cv = jnp.zeros((L,), _i32)
    for b in range(nbk):
        for j in range(CH2 // L):
            sl = pl.ds(j * L, L)
            m = (j * L + iota) >= fills[b]
            st_p[b][sl] = jnp.where(m, 0, st_p[b][sl])
            st_v[b][sl] = jnp.where(m, 0.0, st_v[b][sl])
        rb = (wid * nbk + b) * cap
        pltpu.sync_copy(st_p[b].at[pl.ds(0, CH2)],
                        rec_hbm.at[pl.ds(rb + curs[b] * CH, CH2)])
        pltpu.sync_copy(st_v[b].at[pl.ds(0, CH2)],
                        vslot_hbm.at[pl.ds(rb + curs[b] * CH, CH2)])
        cv = jnp.where(iota == b, ecnt[b], cv)
    countv[...] = cv
    pltpu.sync_copy(countv, counts_hbm.at[wid])


@functools.lru_cache(maxsize=None)
def _make_part(epad, ereal, nbk, cap):
    scratch = ([
        pltpu.VMEM((2 * CH,), _i32),
        pltpu.VMEM((2 * CH,), _i32),
        pltpu.VMEM((2 * CH,), _f32),
        pltpu.VMEM((L,), _i32),
        pltpu.SemaphoreType.DMA,
    ] + [pltpu.VMEM((4 * CH,), _i32) for _ in range(nbk)]
      + [pltpu.VMEM((4 * CH,), _f32) for _ in range(nbk)])
    nslot = NW * nbk
    return pl.kernel(
        functools.partial(_part_body, epad, ereal, nbk, cap),
        out_type=(jax.ShapeDtypeStruct((nslot * cap,), _i32),
                  jax.ShapeDtypeStruct((nslot * cap,), _f32),
                  jax.ShapeDtypeStruct((NW, L), _i32)),
        mesh=_mesh(),
        scratch_types=scratch,
        compiler_params=_sc_params(),
    )


def _slot_count(cref, t, bucket):
    crow = cref[t, pl.ds(0, L)]
    ecnt = jnp.sum(jnp.where(_iota() == bucket, crow, 0))
    return ecnt, ecnt // CH + 1


def _spmm2_body(n_out, nbk, cap, x_hbm, rec_hbm, vals_hbm, counts_hbm,
                out_hbm, gath0, gath1, ed0, ed1, cv0, cv1, valv0, valv1,
                idx0, idx1, cref, acc, gsem0, gsem1, esem0, esem1):
    c = lax.axis_index("c")
    s = lax.axis_index("s")
    nb = nbk // NC
    zero = jnp.zeros((L,), _f32)
    gaths = (gath0, gath1)
    eds = (ed0, ed1)
    cvs = (cv0, cv1)
    valvs = (valv0, valv1)
    idxs = (idx0, idx1)
    gsems = (gsem0, gsem1)
    esems = (esem0, esem1)
    pltpu.sync_copy(counts_hbm, cref)

    def unpack_cols(p):
        for g in range(CH // L):
            sl = pl.ds(g * L, L)
            cvs[p][sl] = eds[p][sl] & 131071

    for b in range(nb):
        bucket = c + NC * b
        row_base = bucket * BSTR

        def zb_body(i, carry):
            for j in range(D // L):
                gath0[i, pl.ds(j * L, L)] = zero
            return carry

        lax.fori_loop(0, CH, zb_body, 0)
        for k in range(RSL // CH):
            pltpu.sync_copy(gath0, acc.at[pl.ds(s * RSL + k * CH, CH)])
        if RSL % CH:
            pltpu.sync_copy(gath0.at[pl.ds(0, RSL % CH)],
                            acc.at[pl.ds(s * RSL + (RSL // CH) * CH,
                                         RSL % CH)])
        plsc.subcore_barrier()

        for tt in range(NC):
            t = s + NS * tt
            ecnt, nch = _slot_count(cref, t, bucket)
            rb0 = (t * nbk + bucket) * cap

            pltpu.sync_copy(rec_hbm.at[pl.ds(rb0, CH)], eds[0])
            pltpu.sync_copy(vals_hbm.at[pl.ds(rb0, CH)], valvs[0])
            unpack_cols(0)
            pltpu.async_copy(x_hbm.at[cvs[0]], gaths[0], gsems[0])

            def pair_body(kp, carry):
                for off in range(2):
                    cur = 2 * kp + off
                    p = off
                    q = 1 - off

                    @pl.when(cur < nch)
                    def _(p=p, q=q, cur=cur):
                        nxt = cur + 1
                        more = nxt < nch

                        @pl.when(more)
                        def _():
                            pltpu.async_copy(
                                rec_hbm.at[pl.ds(rb0 + nxt * CH, CH)],
                                eds[q], esems[q])
                            pltpu.async_copy(
                                vals_hbm.at[pl.ds(rb0 + nxt * CH, CH)],
                                valvs[q], esems[q])

                        for g in range(CH // L):
                            sl = pl.ds(g * L, L)
                            idxs[p][sl] = lax.shift_right_logical(eds[p][sl],
                                                                  17)
                        pltpu.make_async_copy(x_hbm.at[cvs[p]], gaths[p],
                                              gsems[p]).wait()

                        @pl.when(more)
                        def _():
                            pltpu.make_async_copy(
                                rec_hbm.at[pl.ds(rb0 + nxt * CH, CH)],
                                eds[q], esems[q]).wait()
                            pltpu.make_async_copy(
                                vals_hbm.at[pl.ds(rb0 + nxt * CH, CH)],
                                valvs[q], esems[q]).wait()
                            unpack_cols(q)
                            pltpu.async_copy(x_hbm.at[cvs[q]], gaths[q],
                                             gsems[q])

                        def scale_body(ts_, _):
                            for u in range(8):
                                e = ts_ * 8 + u
                                v = plsc.load_gather(
                                    valvs[p], [jnp.full((L,), e, _i32)])
                                for j in range(D // L):
                                    sl = pl.ds(j * L, L)
                                    gaths[p][e, sl] = gaths[p][e, sl] * v
                            return _

                        lax.fori_loop(0, CH // 8, scale_body, 0)
                        pltpu.sync_copy(gaths[p], acc.at[idxs[p]], add=True)
                return carry

            lax.fori_loop(0, (nch + 1) // 2, pair_body, 0)
        plsc.subcore_barrier()

        short_c = (nbk - 1) % NC
        is_short_b = (b == (nbk - 1) // NC)
        tail = n_out - (nbk - 1) * BSTR - 15 * FCH

        def _full_flush():
            pltpu.sync_copy(acc.at[pl.ds(s * FCH, FCH)],
                            out_hbm.at[pl.ds(row_base + s * FCH, FCH)])

        if is_short_b:
            @pl.when((c != short_c) | (s < 15))
            def _():
                _full_flush()

            @pl.when((c == short_c) & (s == 15))
            def _():
                pltpu.sync_copy(acc.at[pl.ds(15 * FCH, tail)],
                                out_hbm.at[pl.ds(row_base + 15 * FCH, tail)])
        else:
            _full_flush()
        plsc.subcore_barrier()


@functools.lru_cache(maxsize=None)
def _make_spmm2(n_x, n_out, nbk, cap):
    scratch = [
        pltpu.VMEM((CH, D), _f32),
        pltpu.VMEM((CH, D), _f32),
        pltpu.VMEM((CH,), _i32),
        pltpu.VMEM((CH,), _i32),
        pltpu.VMEM((CH,), _i32),
        pltpu.VMEM((CH,), _i32),
        pltpu.VMEM((CH,), _f32),
        pltpu.VMEM((CH,), _f32),
        pltpu.VMEM((CH,), _i32),
        pltpu.VMEM((CH,), _i32),
        pltpu.VMEM((NW, L), _i32),
        pltpu.VMEM_SHARED((BACC, D), _f32),
        pltpu.SemaphoreType.DMA,
        pltpu.SemaphoreType.DMA,
        pltpu.SemaphoreType.DMA,
        pltpu.SemaphoreType.DMA,
    ]
    return pl.kernel(
        functools.partial(_spmm2_body, n_out, nbk, cap),
        out_type=jax.ShapeDtypeStruct((n_out, D), _f32),
        mesh=_mesh(),
        scratch_types=scratch,
        compiler_params=_sc_params(),
    )


def _sims2_body(nbk, cap, an_hbm, rec_hbm, counts_hbm, sims_hbm, part_hbm,
                gA, gB, edata, rowb, colb, simbuf, accv, cref, sem1, sem2):
    c = lax.axis_index("c")
    s = lax.axis_index("s")
    wid = s * NC + c
    iota = _iota()
    pltpu.sync_copy(counts_hbm, cref)
    accv[...] = jnp.zeros((L,), _f32)

    for b in range(nbk):
        ecnt, nch = _slot_count(cref, wid, b)
        rb0 = (wid * nbk + b) * cap

        def chunk_body(i, carry):
            pltpu.sync_copy(rec_hbm.at[pl.ds(rb0 + i * CH, CH)], edata)
            for g in range(CH // L):
                sl = pl.ds(g * L, L)
                pk = edata[sl]
                rowb[sl] = lax.shift_right_logical(pk, 17) + b * BSTR
                colb[sl] = pk & 131071
            d1 = pltpu.async_copy(an_hbm.at[rowb], gA, sem1)
            d2 = pltpu.async_copy(an_hbm.at[colb], gB, sem2)
            d1.wait()
            d2.wait()
            for g in range(CH // L):
                F = jnp.zeros((L,), _f32)
                for u in range(L):
                    e = g * L + u
                    p = gA[e, pl.ds(0, L)] * gB[e, pl.ds(0, L)]
                    for j in range(1, D // L):
                        sl = pl.ds(j * L, L)
                        p = p + gA[e, sl] * gB[e, sl]
                    F = jnp.where(iota == u, jnp.sum(p), F)
                sim = (F + 1.0) * 0.5
                simbuf[pl.ds(g * L, L)] = sim
                lid = i * CH + g * L + iota
                accv[...] = accv[...] + jnp.where(lid < ecnt, sim, 0.0)
            pltpu.sync_copy(simbuf, sims_hbm.at[pl.ds(rb0 + i * CH, CH)])
            return carry

        lax.fori_loop(0, nch, chunk_body, 0)
    pltpu.sync_copy(accv, part_hbm.at[wid])


@functools.lru_cache(maxsize=None)
def _make_sims2(nbk, cap):
    scratch = [
        pltpu.VMEM((CH, D), _f32),
        pltpu.VMEM((CH, D), _f32),
        pltpu.VMEM((CH,), _i32),
        pltpu.VMEM((CH,), _i32),
        pltpu.VMEM((CH,), _i32),
        pltpu.VMEM((CH,), _f32),
        pltpu.VMEM((L,), _f32),
        pltpu.VMEM((NW, L), _i32),
        pltpu.SemaphoreType.DMA,
        pltpu.SemaphoreType.DMA,
    ]
    nslot = NW * nbk
    return pl.kernel(
        functools.partial(_sims2_body, nbk, cap),
        out_type=(jax.ShapeDtypeStruct((nslot * cap,), _f32),
                  jax.ShapeDtypeStruct((NW, L), _f32)),
        mesh=_mesh(),
        scratch_types=scratch,
        compiler_params=_sc_params(),
    )


def _mean_threshold(partv):
    tot = jnp.zeros((L,), _f32)
    for t in range(NW):
        tot = tot + partv[t, pl.ds(0, L)]
    mean = jnp.sum(tot) * (1.0 / E_S)
    return jnp.where(mean > 0.7, 0.8, 0.0)


def _diags2_body(nbk, cap, sims_hbm, rec_hbm, counts_hbm, part_hbm, diags_hbm,
                 partv, bins, edata, simv, tmp, accb, cref, stage):
    c = lax.axis_index("c")
    s = lax.axis_index("s")
    iota = _iota()
    pltpu.sync_copy(counts_hbm, cref)
    pltpu.sync_copy(part_hbm, partv)
    prun = _mean_threshold(partv)

    def zero_body(i, carry):
        bins[pl.ds(i * L, L)] = jnp.zeros((L,), _f32)
        return carry

    lax.fori_loop(0, BACC // L, zero_body, 0)

    for tt in range(NC):
        t = s + NS * tt
        ecnt, nch = _slot_count(cref, t, c)
        rb0 = (t * nbk + c) * cap

        def chunk_body(i, carry):
            pltpu.sync_copy(rec_hbm.at[pl.ds(rb0 + i * CH, CH)], edata)
            pltpu.sync_copy(sims_hbm.at[pl.ds(rb0 + i * CH, CH)], simv)
            for g in range(CH // L):
                sv = simv[pl.ds(g * L, L)]
                idx = lax.shift_right_logical(edata[pl.ds(g * L, L)], 17)
                lid = i * CH + g * L + iota
                pruned = jnp.where(sv < prun, 0.0, sv)
                val = jnp.where(lid < ecnt, pruned, 0.0)
                plsc.addupdate_scatter(bins, [idx], val)
            return carry

        lax.fori_loop(0, nch, chunk_body, 0)
    pltpu.sync_copy(bins, stage.at[s])
    plsc.subcore_barrier()

    def zero_acc(i, carry):
        accb[pl.ds(i * L, L)] = jnp.zeros((L,), _f32)
        return carry

    lax.fori_loop(0, RSL // L, zero_acc, 0)

    def red_body(t, carry):
        pltpu.sync_copy(stage.at[t, pl.ds(s * RSL, RSL)], tmp)

        def add_body(r, carry2):
            sl = pl.ds(r * L, L)
            accb[sl] = accb[sl] + tmp[sl]
            return carry2

        lax.fori_loop(0, RSL // L, add_body, 0)
        return carry

    lax.fori_loop(0, NS, red_body, 0)

    def recip_body(r, carry):
        sl = pl.ds(r * L, L)
        accb[sl] = 1.0 / (accb[sl] + 1e-7)
        return carry

    lax.fori_loop(0, RSL // L, recip_body, 0)

    @pl.when(s < 15)
    def _():
        pltpu.sync_copy(accb, diags_hbm.at[c, pl.ds(s * RSL, RSL)])

    @pl.when((s == 15) & (c == 0))
    def _():
        pltpu.sync_copy(accb.at[pl.ds(0, BSTR - 15 * RSL)],
                        diags_hbm.at[c, pl.ds(15 * RSL, BSTR - 15 * RSL)])

    @pl.when((s == 15) & (c == 1))
    def _():
        pltpu.sync_copy(accb.at[pl.ds(0, U_N - BSTR - 15 * RSL)],
                        diags_hbm.at[c, pl.ds(15 * RSL, U_N - BSTR - 15 * RSL)])


@functools.lru_cache(maxsize=None)
def _make_diags2(nbk, cap):
    scratch = [
        pltpu.VMEM((NW, L), _f32),
        pltpu.VMEM((BACC,), _f32),
        pltpu.VMEM((CH,), _i32),
        pltpu.VMEM((CH,), _f32),
        pltpu.VMEM((RSL,), _f32),
        pltpu.VMEM((RSL,), _f32),
        pltpu.VMEM((NW, L), _i32),
        pltpu.VMEM_SHARED((NS, BACC), _f32),
    ]
    return pl.kernel(
        functools.partial(_diags2_body, nbk, cap),
        out_type=jax.ShapeDtypeStruct((NC, BSTR), _f32),
        mesh=_mesh(),
        scratch_types=scratch,
        compiler_params=_sc_params(),
    )


def _nvals2_body(nbk, cap, sims_hbm, rec_hbm, counts_hbm, diags_hbm, part_hbm,
                 out_hbm, dbuf, partv, edata, simv, outv, cref):
    c = lax.axis_index("c")
    s = lax.axis_index("s")
    wid = s * NC + c
    iota = _iota()
    pltpu.sync_copy(counts_hbm, cref)
    pltpu.sync_copy(part_hbm, partv)
    prun = _mean_threshold(partv)
    pltpu.sync_copy(diags_hbm, dbuf)

    for b in range(nbk):
        ecnt, nch = _slot_count(cref, wid, b)
        rb0 = (wid * nbk + b) * cap
        bsplat = jnp.full((L,), b, _i32)

        def chunk_body(i, carry):
            pltpu.sync_copy(rec_hbm.at[pl.ds(rb0 + i * CH, CH)], edata)
            pltpu.sync_copy(sims_hbm.at[pl.ds(rb0 + i * CH, CH)], simv)
            for g in range(CH // L):
                sv = simv[pl.ds(g * L, L)]
                li = lax.shift_right_logical(edata[pl.ds(g * L, L)], 17)
                lid = i * CH + g * L + iota
                pruned = jnp.where(sv < prun, 0.0, sv)
                pruned = jnp.where(lid < ecnt, pruned, 0.0)
                dg = plsc.load_gather(dbuf, [bsplat, li])
                outv[pl.ds(g * L, L)] = pruned * dg
            pltpu.sync_copy(outv, out_hbm.at[pl.ds(rb0 + i * CH, CH)])
            return carry

        lax.fori_loop(0, nch, chunk_body, 0)


@functools.lru_cache(maxsize=None)
def _make_nvals2(nbk, cap):
    scratch = [
        pltpu.VMEM((NC, BSTR), _f32),
        pltpu.VMEM((NW, L), _f32),
        pltpu.VMEM((CH,), _i32),
        pltpu.VMEM((CH,), _f32),
        pltpu.VMEM((CH,), _f32),
        pltpu.VMEM((NW, L), _i32),
    ]
    nslot = NW * nbk
    return pl.kernel(
        functools.partial(_nvals2_body, nbk, cap),
        out_type=jax.ShapeDtypeStruct((nslot * cap,), _f32),
        mesh=_mesh(),
        scratch_types=scratch,
        compiler_params=_sc_params(),
    )


_RB = 1000


def _norm_body(x_ref, o_ref):
    x = x_ref[...]
    n = jnp.sqrt(jnp.sum(x * x, axis=1, keepdims=True))
    o_ref[...] = x / jnp.maximum(n, 1e-8)


def _normalize(x):
    return pl.pallas_call(
        _norm_body,
        grid=(U_N // _RB,),
        in_specs=[pl.BlockSpec((_RB, D), lambda i: (i, 0))],
        out_specs=pl.BlockSpec((_RB, D), lambda i: (i, 0)),
        out_shape=jax.ShapeDtypeStruct((U_N, D), _f32),
    )(x)


def _ego_body(u_ref, l1_ref, l2_ref, l3_ref, it_ref, ego_ref, sv_ref):
    i = pl.program_id(0)
    sv = (l1_ref[...] + l2_ref[...] + l3_ref[...]) * (1.0 / 3.0)
    sv_ref[...] = sv

    @pl.when(i < U_N // _RB)
    def _():
        ego_ref[...] = u_ref[...] + sv

    @pl.when(i >= U_N // _RB)
    def _():
        ego_ref[...] = it_ref[...]


def _ego(user_emb, l1, l2, l3, item_emb):
    nu = U_N // _RB
    user_map = lambda i: (jnp.minimum(i, nu - 1), 0)
    item_map = lambda i: (jnp.maximum(i - nu, 0), 0)
    return pl.pallas_call(
        _ego_body,
        grid=((U_N + I_N) // _RB,),
        in_specs=[
            pl.BlockSpec((_RB, D), user_map),
            pl.BlockSpec((_RB, D), user_map),
            pl.BlockSpec((_RB, D), user_map),
            pl.BlockSpec((_RB, D), user_map),
            pl.BlockSpec((_RB, D), item_map),
        ],
        out_specs=(
            pl.BlockSpec((_RB, D), lambda i: (i, 0)),
            pl.BlockSpec((_RB, D), user_map),
        ),
        out_shape=(jax.ShapeDtypeStruct((U_N + I_N, D), _f32),
                   jax.ShapeDtypeStruct((U_N, D), _f32)),
    )(user_emb, l1, l2, l3, item_emb)


def _final_body(e0u_ref, e1u_ref, e2u_ref, e0i_ref, e1i_ref, e2i_ref,
                sv_ref, g1_ref, g2_ref, user_ref, item_ref):
    uv1 = (e0u_ref[...] + e1u_ref[...] + e2u_ref[...]) * (1.0 / 3.0)
    iv1 = (e0i_ref[...] + e1i_ref[...] + e2i_ref[...]) * (1.0 / 3.0)
    item_ref[...] = iv1
    sv = sv_ref[...]
    dn = (((1,), (1,)), ((), ()))
    z = (lax.dot_general(uv1, g1_ref[...], dn, preferred_element_type=_f32)
         + lax.dot_general(sv, g2_ref[...], dn, preferred_element_type=_f32))
    g = jax.nn.sigmoid(z)
    user_ref[...] = g * sv + (1.0 - g) * uv1


def _final(ego, e1, e2, sview, g1, g2):
    nu = U_N // _RB
    umap = lambda i: (i, 0)
    imap = lambda i: (i + nu, 0)
    wmap = lambda i: (0, 0)
    return pl.pallas_call(
        _final_body,
        grid=(nu,),
        in_specs=[
            pl.BlockSpec((_RB, D), umap),
            pl.BlockSpec((_RB, D), umap),
            pl.BlockSpec((_RB, D), umap),
            pl.BlockSpec((_RB, D), imap),
            pl.BlockSpec((_RB, D), imap),
            pl.BlockSpec((_RB, D), imap),
            pl.BlockSpec((_RB, D), umap),
            pl.BlockSpec((D, D), wmap),
            pl.BlockSpec((D, D), wmap),
        ],
        out_specs=(pl.BlockSpec((_RB, D), umap),
                   pl.BlockSpec((_RB, D), umap)),
        out_shape=(jax.ShapeDtypeStruct((U_N, D), _f32),
                   jax.ShapeDtypeStruct((U_N, D), _f32)),
    )(ego, e1, e2, ego, e1, e2, sview, g1, g2)


def kernel(user_emb, item_emb, gate1_W, gate2_W, social_index, social_values,
           norm_index, norm_values):
    s_rows = jnp.pad(social_index[0], (0, ES_P - E_S))
    s_cols = jnp.pad(social_index[1], (0, ES_P - E_S))
    s_vals = jnp.pad(social_values, (0, ES_P - E_S))
    u_rows = jnp.pad(norm_index[0], (0, EU_P - E_UI))
    u_cols = jnp.pad(norm_index[1], (0, EU_P - E_UI))
    u_vals = jnp.pad(norm_values, (0, EU_P - E_UI))

    rec_s, vslot_s, cnt_s = _make_part(ES_P, E_S, NBK_S, CAP_S)(
        s_rows, s_cols, s_vals)
    rec_u, vslot_u, cnt_u = _make_part(EU_P, E_UI, NBK_UI, CAP_UI)(
        u_rows, u_cols, u_vals)

    spmm_s = _make_spmm2(U_N, U_N, NBK_S, CAP_S)
    spmm_ui = _make_spmm2(U_N + I_N, U_N + I_N, NBK_UI, CAP_UI)

    u_agg = spmm_s(user_emb, rec_s, vslot_s, cnt_s)
    a_norm = _normalize(u_agg)
    sims, parts = _make_sims2(NBK_S, CAP_S)(a_norm, rec_s, cnt_s)
    diags = _make_diags2(NBK_S, CAP_S)(sims, rec_s, cnt_s, parts)
    nvals = _make_nvals2(NBK_S, CAP_S)(sims, rec_s, cnt_s, diags, parts)
    l1 = spmm_s(user_emb, rec_s, nvals, cnt_s)
    l2 = spmm_s(l1, rec_s, nvals, cnt_s)
    l3 = spmm_s(l2, rec_s, nvals, cnt_s)
    ego, sview = _ego(user_emb, l1, l2, l3, item_emb)
    e1 = spmm_ui(ego, rec_u, vslot_u, cnt_u)
    e2 = spmm_ui(e1, rec_u, vslot_u, cnt_u)
    return _final(ego, e1, e2, sview, gate1_W, gate2_W)
